# baseline (speedup 1.0000x reference)
import functools

import jax
import jax.numpy as jnp
from jax import lax
from jax.experimental import pallas as pl
from jax.experimental.pallas import tpu as pltpu
from jax.experimental.pallas import tpu_sc as plsc

_B = 64
_BEAM = 4
_V = 100000
_CUR = 8
_EOS = 2
_NEG_INF = -1e20
_ROW = _BEAM * _V
_LANES = 128
_SUB = _ROW // _LANES

_NW_WORKERS = 32
_BPW = _B // _NW_WORKERS
_CHUNK = 20000
_NCHUNK = _BPW * _ROW // _CHUNK
_WIN = 2000
_NWIN = _ROW // _WIN
_FILL = -1.0e30
_BIGI = 2 ** 30
_CBUF = 512


def _sc_topk_body(scores_hbm, alive_hbm, vals_hbm, idx_hbm,
                  buf0, buf1, mw, rbuf, cval, cidx, asv, outv, outi,
                  sem0, sem1):
    wid = lax.axis_index("s") * 2 + lax.axis_index("c")
    base = wid * (_BPW * _ROW)

    pltpu.sync_copy(alive_hbm.at[pl.ds(wid * (_BPW * _BEAM), _BPW * _BEAM)],
                    asv.at[pl.ds(0, _BPW * _BEAM)])
    av = asv[...]

    def _chunk_src(c):
        return scores_hbm.at[pl.ds(base + c * _CHUNK, _CHUNK)]

    def _compute_chunk(buf, chunk_i):
        for j in range(_CHUNK // _WIN):
            def inner(t, accs):
                o = j * _WIN + t * 80
                return tuple(
                    jnp.maximum(a, buf[pl.ds(o + 16 * q, 16)])
                    for q, a in enumerate(accs))
            init = tuple(jnp.full((16,), _FILL, jnp.float32) for _ in range(5))
            a0, a1, a2, a3, a4 = lax.fori_loop(0, _WIN // 80, inner, init)
            acc = jnp.maximum(jnp.maximum(jnp.maximum(a0, a1),
                                          jnp.maximum(a2, a3)), a4)
            mw[pl.ds((chunk_i * (_CHUNK // _WIN) + j) * 16, 16)] = acc

    pltpu.async_copy(_chunk_src(0), buf0, sem0)

    def outer(i, carry):
        pltpu.async_copy(_chunk_src(2 * i + 1), buf1, sem1)
        pltpu.make_async_copy(_chunk_src(0), buf0, sem0).wait()
        _compute_chunk(buf0, 2 * i)

        @pl.when(i < _NCHUNK // 2 - 1)
        def _():
            pltpu.async_copy(_chunk_src(2 * i + 2), buf0, sem0)

        pltpu.make_async_copy(_chunk_src(1), buf1, sem1).wait()
        _compute_chunk(buf1, 2 * i + 1)
        return carry

    lax.fori_loop(0, _NCHUNK // 2, outer, jnp.int32(0))

    lane = lax.iota(jnp.int32, 16)
    ovec = jnp.full((16,), _FILL, jnp.float32)
    oivec = jnp.full((16,), 0, jnp.int32)

    for lb in range(_BPW):
        mwbase = lb * _NWIN * 16
        s0 = av[lb * _BEAM + 0]
        s1 = av[lb * _BEAM + 1]
        s2 = av[lb * _BEAM + 2]
        s3 = av[lb * _BEAM + 3]
        wps = _V // _WIN

        def _sk(wi):
            return jnp.where(wi < wps, s0,
                             jnp.where(wi < 2 * wps, s1,
                                       jnp.where(wi < 3 * wps, s2, s3)))

        def tins(wi, carry):
            t1, t2, t3, t4 = carry
            x = mw[pl.ds(mwbase + wi * 16, 16)] + _sk(wi)
            d = jnp.minimum(t1, x)
            t1 = jnp.maximum(t1, x)
            d2 = jnp.minimum(t2, d)
            t2 = jnp.maximum(t2, d)
            d3 = jnp.minimum(t3, d2)
            t3 = jnp.maximum(t3, d2)
            t4 = jnp.maximum(t4, d3)
            return (t1, t2, t3, t4)

        fill = jnp.full((16,), _FILL, jnp.float32)
        t1, t2, t3, t4 = lax.fori_loop(0, _NWIN, tins, (fill, fill, fill, fill))

        T = jnp.float32(0)
        for r in range(4):
            mm = jnp.maximum(jnp.maximum(t1, t2), jnp.maximum(t3, t4))
            T = jnp.max(mm)
            if r < 3:
                t1 = jnp.where(t1 == T, _FILL, t1)
                t2 = jnp.where(t2 == T, _FILL, t2)
                t3 = jnp.where(t3 == T, _FILL, t3)
                t4 = jnp.where(t4 == T, _FILL, t4)

        for q in range(_CBUF // 16):
            cval[pl.ds(q * 16, 16)] = fill
            cidx[pl.ds(q * 16, 16)] = jnp.full((16,), _BIGI, jnp.int32)

        def rw(wi, off):
            a = mw[pl.ds(mwbase + wi * 16, 16)]
            sk = _sk(wi)
            wmax = jnp.max(a) + sk

            def do_rescan(off):
                pltpu.sync_copy(
                    scores_hbm.at[pl.ds(base + lb * _ROW + wi * _WIN, _WIN)],
                    rbuf)

                def rv(t, off):
                    y = rbuf[pl.ds(t * 16, 16)] + sk
                    msk = y >= T
                    iv = lane + (wi * _WIN + t * 16)
                    plsc.store_compressed(cval.at[pl.ds(off, 16)], y, mask=msk)
                    plsc.store_compressed(cidx.at[pl.ds(off, 16)], iv, mask=msk)
                    cnt = plsc.all_reduce_population_count(msk)
                    return jnp.minimum(off + jnp.max(cnt),
                                       jnp.int32(_CBUF - 16))

                return lax.fori_loop(0, _WIN // 16, rv, off)

            return lax.cond(wmax >= T, do_rescan, lambda o: o, off)

        lax.fori_loop(0, _NWIN, rw, jnp.int32(0))

        for r in range(4):
            def scan_best(q, carry):
                vb, vi = carry
                v = cval[pl.ds(q * 16, 16)]
                ix = cidx[pl.ds(q * 16, 16)]
                better = (v > vb) | ((v == vb) & (ix < vi))
                return (jnp.where(better, v, vb), jnp.where(better, ix, vi))

            vb, vi = lax.fori_loop(
                0, _CBUF // 16, scan_best,
                (fill, jnp.full((16,), _BIGI, jnp.int32)))
            m = jnp.max(vb)
            mi = jnp.min(jnp.where(vb == m, vi, _BIGI))

            def rem(q, carry):
                ix = cidx[pl.ds(q * 16, 16)]
                v = cval[pl.ds(q * 16, 16)]
                cval[pl.ds(q * 16, 16)] = jnp.where(ix == mi, _FILL, v)
                return carry

            lax.fori_loop(0, _CBUF // 16, rem, jnp.int32(0))
            pos = lb * _BEAM + r
            ovec = jnp.where(lane == pos, m, ovec)
            oivec = jnp.where(lane == pos, mi, oivec)

    outv[...] = ovec
    outi[...] = oivec
    n_out = _BPW * _BEAM
    pltpu.sync_copy(outv.at[pl.ds(0, n_out)], vals_hbm.at[pl.ds(wid * n_out, n_out)])
    pltpu.sync_copy(outi.at[pl.ds(0, n_out)], idx_hbm.at[pl.ds(wid * n_out, n_out)])


def _make_sc_topk():
    return functools.partial(
        pl.kernel,
        mesh=plsc.VectorSubcoreMesh(core_axis_name="c", subcore_axis_name="s"),
        compiler_params=pltpu.CompilerParams(needs_layout_passes=False),
        out_type=[
        jax.ShapeDtypeStruct((_B * _BEAM,), jnp.float32),
        jax.ShapeDtypeStruct((_B * _BEAM,), jnp.int32),
    ],
    scratch_types=[
        pltpu.VMEM((_CHUNK,), jnp.float32),
        pltpu.VMEM((_CHUNK,), jnp.float32),
        pltpu.VMEM((_BPW * _NWIN * 16,), jnp.float32),
        pltpu.VMEM((_WIN,), jnp.float32),
        pltpu.VMEM((_CBUF,), jnp.float32),
        pltpu.VMEM((_CBUF,), jnp.int32),
        pltpu.VMEM((16,), jnp.float32),
        pltpu.VMEM((16,), jnp.float32),
        pltpu.VMEM((16,), jnp.int32),
            pltpu.SemaphoreType.DMA,
            pltpu.SemaphoreType.DMA,
        ],
    )(_sc_topk_body)


_CHL = 2048
_NCHL = 49
_TAIL_A = 48 * _CHL
_TFILL = -3.0e38
_G = 4
_BIG = 2 ** 30


def _scan_body(x_ref, alive_ref, cmax_ref):
    i0 = pl.program_id(0) * _G
    br = jax.lax.broadcasted_iota(jnp.int32, (_BEAM, 1), 0)
    for g in range(_G):
        s_list = [alive_ref[i0 + g, k] for k in range(_BEAM)]
        s_col = jnp.where(br == 0, s_list[0],
                          jnp.where(br == 1, s_list[1],
                                    jnp.where(br == 2, s_list[2], s_list[3])))
        cms = []
        for c in range(_NCHL):
            a = c * _CHL
            if c < _NCHL - 1:
                w = x_ref[g, :, a:a + 128]
                for t in range(1, _CHL // 128):
                    w = jnp.maximum(w, x_ref[g, :, a + 128 * t:a + 128 * (t + 1)])
                cm = jnp.max(w, axis=1, keepdims=True)
            else:
                w = x_ref[g, :, a:a + 128]
                for t in range(1, (_V - _TAIL_A) // 128):
                    w = jnp.maximum(w, x_ref[g, :, a + 128 * t:a + 128 * (t + 1)])
                cm = jnp.maximum(
                    jnp.max(w, axis=1, keepdims=True),
                    jnp.max(x_ref[g, :, _TAIL_A + ((_V - _TAIL_A) // 128) * 128:_V],
                            axis=1, keepdims=True))
            cms.append(cm)
        cmat = jnp.concatenate(cms, axis=1) + s_col
        cmax_ref[0, g:g + 1, :] = jnp.max(cmat, axis=0, keepdims=True)


def _chunksel_body(cmax_ref, cand_ref):
    cm = cmax_ref[...]
    cio = jax.lax.broadcasted_iota(jnp.int32, (_B, _NCHL), 1)
    ids = []
    for r in range(_BEAM):
        m = jnp.max(cm, axis=1, keepdims=True)
        cid = jnp.min(jnp.where(cm == m, cio, _BIG), axis=1, keepdims=True)
        ids.append(cid)
        cm = jnp.where(cio == cid, _TFILL, cm)
    cand_ref[...] = jnp.concatenate(ids, axis=1)


def _gather_body(cand_ref, x_ref, o_ref):
    o_ref[0, 0] = x_ref[0]


def _extract_body(gy_ref, cand_ref, alive_ref, vals_ref, idx_ref):
    gy = gy_ref[...]
    shp = (_B, 4, _BEAM, _CHL)
    jio = jax.lax.broadcasted_iota(jnp.int32, shp, 1)
    kio = jax.lax.broadcasted_iota(jnp.int32, shp, 2)
    lio = jax.lax.broadcasted_iota(jnp.int32, shp, 3)
    cid_e = jnp.zeros(shp, jnp.int32)
    s_e = jnp.zeros(shp, jnp.float32)
    for j in range(4):
        cid_e = jnp.where(jio == j, cand_ref[:, j:j + 1, :, :], cid_e)
    for k in range(_BEAM):
        s_e = jnp.where(kio == k, alive_ref[:, k:k + 1, :, :], s_e)
    vpos = cid_e * _CHL + lio
    valid = vpos < _V
    y = gy + s_e
    my = jnp.where(valid, y, _TFILL)
    fidx = kio * _V + vpos
    vs = []
    ix = []
    for r in range(_BEAM):
        m = jnp.max(my, axis=3, keepdims=True)
        m = jnp.max(m, axis=2, keepdims=True)
        m = jnp.max(m, axis=1, keepdims=True)
        c1 = jnp.min(jnp.where(my == m, fidx, _BIG), axis=3, keepdims=True)
        c1 = jnp.min(c1, axis=2, keepdims=True)
        sel = jnp.min(c1, axis=1, keepdims=True)
        vs.append(m)
        ix.append(sel)
        my = jnp.where(fidx == sel, _TFILL, my)
    vals_ref[...] = jnp.concatenate(vs, axis=1)
    idx_ref[...] = jnp.concatenate(ix, axis=1)


def _topk_body(score_ref, alive_ref, vals_ref, idx_ref):
    b = pl.program_id(0)
    x = score_ref[0]
    ridx = jax.lax.broadcasted_iota(jnp.int32, (_SUB, _LANES), 0)
    cidx = jax.lax.broadcasted_iota(jnp.int32, (_SUB, _LANES), 1)
    idx = ridx * _LANES + cidx
    s0 = alive_ref[b, 0]
    s1 = alive_ref[b, 1]
    s2 = alive_ref[b, 2]
    s3 = alive_ref[b, 3]
    add = jnp.where(idx < _V, s0, jnp.where(idx < 2 * _V, s1,
                    jnp.where(idx < 3 * _V, s2, s3)))
    y = x + add
    big = jnp.int32(2 ** 30)
    for r in range(_BEAM):
        m = jnp.max(y)
        sel = jnp.min(jnp.where(y == m, idx, big))
        vals_ref[0, 0, r] = m
        idx_ref[0, 0, r] = sel
        y = jnp.where(idx == sel, _NEG_INF, y)


def _finish_body(vals_ref, idx_ref, hyp_ref, ts_ref, as_ref, fm_ref, tok_ref, hyp_out_ref):
    top_scores = vals_ref[:, 0, :]
    index = idx_ref[:, 0, :]
    tokens = index % _V
    origin = index // _V
    hyp = hyp_ref[...]
    lane = jax.lax.broadcasted_iota(jnp.int32, (_B, _BEAM * _CUR), 1)
    grp = lane // _CUR
    zero32 = jnp.zeros((_B, _BEAM * _CUR), jnp.int32)
    origin_e = zero32
    for j in range(_BEAM):
        origin_e = jnp.where(grp == j, origin[:, j:j + 1], origin_e)
    cand = zero32
    for k in range(_BEAM):
        tile_k = jnp.concatenate([hyp[:, k * _CUR:(k + 1) * _CUR]] * _BEAM, axis=1)
        cand = jnp.where(origin_e == k, tile_k, cand)
    flags = (tokens == _EOS).astype(jnp.float32)
    alive_masked = top_scores + flags * _NEG_INF
    finish_masked = top_scores + (1.0 - flags) * _NEG_INF
    iota4 = jax.lax.broadcasted_iota(jnp.int32, (_B, _BEAM), 1)
    am = alive_masked
    new_scores = []
    new_idx = []
    for r in range(_BEAM):
        m = jnp.max(am, axis=1, keepdims=True)
        sel = jnp.min(jnp.where(am == m, iota4, _BEAM), axis=1, keepdims=True)
        new_scores.append(m)
        new_idx.append(sel)
        am = jnp.where(iota4 == sel, _NEG_INF, am)
    alive_scores_new = jnp.concatenate(new_scores, axis=1)
    alive_idx = jnp.concatenate(new_idx, axis=1)
    aidx_e = zero32
    for j in range(_BEAM):
        aidx_e = jnp.where(grp == j, alive_idx[:, j:j + 1], aidx_e)
    new_hyp = zero32
    new_tok = jnp.zeros((_B, _BEAM), jnp.int32)
    for k in range(_BEAM):
        tile_k = jnp.concatenate([cand[:, k * _CUR:(k + 1) * _CUR]] * _BEAM, axis=1)
        new_hyp = jnp.where(aidx_e == k, tile_k, new_hyp)
        new_tok = jnp.where(alive_idx == k, tokens[:, k:k + 1], new_tok)
    ts_ref[...] = top_scores
    as_ref[...] = alive_scores_new
    fm_ref[...] = finish_masked
    tok_ref[...] = tokens
    hyp_out_ref[...] = jnp.concatenate(
        [jnp.concatenate([new_hyp[:, j * _CUR:(j + 1) * _CUR],
                          new_tok[:, j:j + 1]], axis=1)
         for j in range(_BEAM)], axis=1)


def kernel(out, alive_scores, alive_hypotheses):
    cmax = pl.pallas_call(
        _scan_body,
        grid=(_B // _G,),
        in_specs=[
            pl.BlockSpec((_G, _BEAM, _V), lambda b: (b, 0, 0)),
            pl.BlockSpec(memory_space=pltpu.SMEM),
        ],
        out_specs=pl.BlockSpec((1, _G, _NCHL), lambda b: (b, 0, 0)),
        out_shape=jax.ShapeDtypeStruct((_B // _G, _G, _NCHL), jnp.float32),
    )(out, alive_scores)

    cand = pl.pallas_call(
        _chunksel_body,
        out_shape=jax.ShapeDtypeStruct((_B, _BEAM), jnp.int32),
    )(cmax.reshape(_B, _NCHL))

    gathered = pl.pallas_call(
        _gather_body,
        grid_spec=pltpu.PrefetchScalarGridSpec(
            num_scalar_prefetch=1,
            grid=(_B, 4),
            in_specs=[
                pl.BlockSpec((1, _BEAM, _CHL),
                             lambda b, j, cand_pref: (b, 0, cand_pref[b, j])),
            ],
            out_specs=pl.BlockSpec((1, 1, _BEAM, _CHL),
                                   lambda b, j, cand_pref: (b, j, 0, 0)),
        ),
        out_shape=jax.ShapeDtypeStruct((_B, 4, _BEAM, _CHL), jnp.float32),
    )(cand, out)

    vals4, idx4 = pl.pallas_call(
        _extract_body,
        out_shape=[
            jax.ShapeDtypeStruct((_B, _BEAM, 1, 1), jnp.float32),
            jax.ShapeDtypeStruct((_B, _BEAM, 1, 1), jnp.int32),
        ],
    )(gathered, cand.reshape(_B, _BEAM, 1, 1),
      alive_scores.reshape(_B, _BEAM, 1, 1))
    vals = vals4.reshape(_B, 1, _BEAM)
    idx = idx4.reshape(_B, 1, _BEAM)

    ts, asn, fm, tok, hyp_new = pl.pallas_call(
        _finish_body,
        out_shape=[
            jax.ShapeDtypeStruct((_B, _BEAM), jnp.float32),
            jax.ShapeDtypeStruct((_B, _BEAM), jnp.float32),
            jax.ShapeDtypeStruct((_B, _BEAM), jnp.float32),
            jax.ShapeDtypeStruct((_B, _BEAM), jnp.int32),
            jax.ShapeDtypeStruct((_B, _BEAM * (_CUR + 1)), jnp.int32),
        ],
    )(vals, idx, alive_hypotheses.reshape(_B, _BEAM * _CUR))
    return (ts, asn, fm, tok, hyp_new.reshape(_B * _BEAM, _CUR + 1))


# speedup vs baseline: 1.4179x; 1.4179x over previous
import functools

import jax
import jax.numpy as jnp
from jax import lax
from jax.experimental import pallas as pl
from jax.experimental.pallas import tpu as pltpu
from jax.experimental.pallas import tpu_sc as plsc

_B = 64
_BEAM = 4
_V = 100000
_CUR = 8
_EOS = 2
_NEG_INF = -1e20
_ROW = _BEAM * _V
_LANES = 128
_SUB = _ROW // _LANES

_NW_WORKERS = 32
_BPW = _B // _NW_WORKERS
_CHUNK = 20000
_NCHUNK = _BPW * _ROW // _CHUNK
_WIN = 2000
_NWIN = _ROW // _WIN
_FILL = -1.0e30
_BIGI = 2 ** 30
_CBUF = 512


def _sc_topk_body(scores_hbm, alive_hbm, vals_hbm, idx_hbm,
                  buf0, buf1, mw, rbuf, cval, cidx, asv, outv, outi,
                  sem0, sem1):
    wid = lax.axis_index("s") * 2 + lax.axis_index("c")
    base = wid * (_BPW * _ROW)

    pltpu.sync_copy(alive_hbm.at[pl.ds(wid * (_BPW * _BEAM), _BPW * _BEAM)],
                    asv.at[pl.ds(0, _BPW * _BEAM)])
    av = asv[...]

    def _chunk_src(c):
        return scores_hbm.at[pl.ds(base + c * _CHUNK, _CHUNK)]

    def _compute_chunk(buf, chunk_i):
        for j in range(_CHUNK // _WIN):
            def inner(t, accs):
                o = j * _WIN + t * 80
                return tuple(
                    jnp.maximum(a, buf[pl.ds(o + 16 * q, 16)])
                    for q, a in enumerate(accs))
            init = tuple(jnp.full((16,), _FILL, jnp.float32) for _ in range(5))
            a0, a1, a2, a3, a4 = lax.fori_loop(0, _WIN // 80, inner, init)
            acc = jnp.maximum(jnp.maximum(jnp.maximum(a0, a1),
                                          jnp.maximum(a2, a3)), a4)
            mw[pl.ds((chunk_i * (_CHUNK // _WIN) + j) * 16, 16)] = acc

    pltpu.async_copy(_chunk_src(0), buf0, sem0)

    def outer(i, carry):
        pltpu.async_copy(_chunk_src(2 * i + 1), buf1, sem1)
        pltpu.make_async_copy(_chunk_src(0), buf0, sem0).wait()
        _compute_chunk(buf0, 2 * i)

        @pl.when(i < _NCHUNK // 2 - 1)
        def _():
            pltpu.async_copy(_chunk_src(2 * i + 2), buf0, sem0)

        pltpu.make_async_copy(_chunk_src(1), buf1, sem1).wait()
        _compute_chunk(buf1, 2 * i + 1)
        return carry

    lax.fori_loop(0, _NCHUNK // 2, outer, jnp.int32(0))

    lane = lax.iota(jnp.int32, 16)
    ovec = jnp.full((16,), _FILL, jnp.float32)
    oivec = jnp.full((16,), 0, jnp.int32)

    for lb in range(_BPW):
        mwbase = lb * _NWIN * 16
        s0 = av[lb * _BEAM + 0]
        s1 = av[lb * _BEAM + 1]
        s2 = av[lb * _BEAM + 2]
        s3 = av[lb * _BEAM + 3]
        wps = _V // _WIN

        def _sk(wi):
            return jnp.where(wi < wps, s0,
                             jnp.where(wi < 2 * wps, s1,
                                       jnp.where(wi < 3 * wps, s2, s3)))

        def tins(wi, carry):
            t1, t2, t3, t4 = carry
            x = mw[pl.ds(mwbase + wi * 16, 16)] + _sk(wi)
            d = jnp.minimum(t1, x)
            t1 = jnp.maximum(t1, x)
            d2 = jnp.minimum(t2, d)
            t2 = jnp.maximum(t2, d)
            d3 = jnp.minimum(t3, d2)
            t3 = jnp.maximum(t3, d2)
            t4 = jnp.maximum(t4, d3)
            return (t1, t2, t3, t4)

        fill = jnp.full((16,), _FILL, jnp.float32)
        t1, t2, t3, t4 = lax.fori_loop(0, _NWIN, tins, (fill, fill, fill, fill))

        T = jnp.float32(0)
        for r in range(4):
            mm = jnp.maximum(jnp.maximum(t1, t2), jnp.maximum(t3, t4))
            T = jnp.max(mm)
            if r < 3:
                t1 = jnp.where(t1 == T, _FILL, t1)
                t2 = jnp.where(t2 == T, _FILL, t2)
                t3 = jnp.where(t3 == T, _FILL, t3)
                t4 = jnp.where(t4 == T, _FILL, t4)

        for q in range(_CBUF // 16):
            cval[pl.ds(q * 16, 16)] = fill
            cidx[pl.ds(q * 16, 16)] = jnp.full((16,), _BIGI, jnp.int32)

        def rw(wi, off):
            a = mw[pl.ds(mwbase + wi * 16, 16)]
            sk = _sk(wi)
            wmax = jnp.max(a) + sk

            def do_rescan(off):
                pltpu.sync_copy(
                    scores_hbm.at[pl.ds(base + lb * _ROW + wi * _WIN, _WIN)],
                    rbuf)

                def rv(t, off):
                    y = rbuf[pl.ds(t * 16, 16)] + sk
                    msk = y >= T
                    iv = lane + (wi * _WIN + t * 16)
                    plsc.store_compressed(cval.at[pl.ds(off, 16)], y, mask=msk)
                    plsc.store_compressed(cidx.at[pl.ds(off, 16)], iv, mask=msk)
                    cnt = plsc.all_reduce_population_count(msk)
                    return jnp.minimum(off + jnp.max(cnt),
                                       jnp.int32(_CBUF - 16))

                return lax.fori_loop(0, _WIN // 16, rv, off)

            return lax.cond(wmax >= T, do_rescan, lambda o: o, off)

        lax.fori_loop(0, _NWIN, rw, jnp.int32(0))

        for r in range(4):
            def scan_best(q, carry):
                vb, vi = carry
                v = cval[pl.ds(q * 16, 16)]
                ix = cidx[pl.ds(q * 16, 16)]
                better = (v > vb) | ((v == vb) & (ix < vi))
                return (jnp.where(better, v, vb), jnp.where(better, ix, vi))

            vb, vi = lax.fori_loop(
                0, _CBUF // 16, scan_best,
                (fill, jnp.full((16,), _BIGI, jnp.int32)))
            m = jnp.max(vb)
            mi = jnp.min(jnp.where(vb == m, vi, _BIGI))

            def rem(q, carry):
                ix = cidx[pl.ds(q * 16, 16)]
                v = cval[pl.ds(q * 16, 16)]
                cval[pl.ds(q * 16, 16)] = jnp.where(ix == mi, _FILL, v)
                return carry

            lax.fori_loop(0, _CBUF // 16, rem, jnp.int32(0))
            pos = lb * _BEAM + r
            ovec = jnp.where(lane == pos, m, ovec)
            oivec = jnp.where(lane == pos, mi, oivec)

    outv[...] = ovec
    outi[...] = oivec
    n_out = _BPW * _BEAM
    pltpu.sync_copy(outv.at[pl.ds(0, n_out)], vals_hbm.at[pl.ds(wid * n_out, n_out)])
    pltpu.sync_copy(outi.at[pl.ds(0, n_out)], idx_hbm.at[pl.ds(wid * n_out, n_out)])


def _make_sc_topk():
    return functools.partial(
        pl.kernel,
        mesh=plsc.VectorSubcoreMesh(core_axis_name="c", subcore_axis_name="s"),
        compiler_params=pltpu.CompilerParams(needs_layout_passes=False),
        out_type=[
        jax.ShapeDtypeStruct((_B * _BEAM,), jnp.float32),
        jax.ShapeDtypeStruct((_B * _BEAM,), jnp.int32),
    ],
    scratch_types=[
        pltpu.VMEM((_CHUNK,), jnp.float32),
        pltpu.VMEM((_CHUNK,), jnp.float32),
        pltpu.VMEM((_BPW * _NWIN * 16,), jnp.float32),
        pltpu.VMEM((_WIN,), jnp.float32),
        pltpu.VMEM((_CBUF,), jnp.float32),
        pltpu.VMEM((_CBUF,), jnp.int32),
        pltpu.VMEM((16,), jnp.float32),
        pltpu.VMEM((16,), jnp.float32),
        pltpu.VMEM((16,), jnp.int32),
            pltpu.SemaphoreType.DMA,
            pltpu.SemaphoreType.DMA,
        ],
    )(_sc_topk_body)


_CHL = 2048
_NCHL = 49
_TAIL_A = 48 * _CHL
_TFILL = -3.0e38
_G = 4
_BIG = 2 ** 30


def _scan_body(x_ref, alive_ref, cmax_ref):
    i0 = pl.program_id(0) * _G
    br = jax.lax.broadcasted_iota(jnp.int32, (_BEAM, 1), 0)
    for g in range(_G):
        s_list = [alive_ref[i0 + g, k] for k in range(_BEAM)]
        s_col = jnp.where(br == 0, s_list[0],
                          jnp.where(br == 1, s_list[1],
                                    jnp.where(br == 2, s_list[2], s_list[3])))
        cms = []
        for c in range(_NCHL):
            a = c * _CHL
            if c < _NCHL - 1:
                w = x_ref[g, :, a:a + 128]
                for t in range(1, _CHL // 128):
                    w = jnp.maximum(w, x_ref[g, :, a + 128 * t:a + 128 * (t + 1)])
                cm = jnp.max(w, axis=1, keepdims=True)
            else:
                w = x_ref[g, :, a:a + 128]
                for t in range(1, (_V - _TAIL_A) // 128):
                    w = jnp.maximum(w, x_ref[g, :, a + 128 * t:a + 128 * (t + 1)])
                cm = jnp.maximum(
                    jnp.max(w, axis=1, keepdims=True),
                    jnp.max(x_ref[g, :, _TAIL_A + ((_V - _TAIL_A) // 128) * 128:_V],
                            axis=1, keepdims=True))
            cms.append(cm)
        cmat = jnp.concatenate(cms, axis=1) + s_col
        cmax_ref[0, g:g + 1, :] = jnp.max(cmat, axis=0, keepdims=True)


def _chunksel_body(cmax_ref, cand_ref):
    cm = cmax_ref[...]
    cio = jax.lax.broadcasted_iota(jnp.int32, (_B, _NCHL), 1)
    ids = []
    for r in range(_BEAM):
        m = jnp.max(cm, axis=1, keepdims=True)
        cid = jnp.min(jnp.where(cm == m, cio, _BIG), axis=1, keepdims=True)
        ids.append(cid)
        cm = jnp.where(cio == cid, _TFILL, cm)
    cand_ref[...] = jnp.concatenate(ids, axis=1)


def _gx_body(cand_ref, x0_ref, x1_ref, x2_ref, x3_ref, alive_ref,
             vals_ref, idx_ref):
    b = pl.program_id(0)
    kio2 = jax.lax.broadcasted_iota(jnp.int32, (_BEAM, _CHL), 0) * _V
    lio2 = jax.lax.broadcasted_iota(jnp.int32, (_BEAM, _CHL), 1)
    br = jax.lax.broadcasted_iota(jnp.int32, (_BEAM, 1), 0)
    s_list = [alive_ref[b, k] for k in range(_BEAM)]
    s_col = jnp.where(br == 0, s_list[0],
                      jnp.where(br == 1, s_list[1],
                                jnp.where(br == 2, s_list[2], s_list[3])))
    mys = []
    fids = []
    for j, xr in enumerate((x0_ref, x1_ref, x2_ref, x3_ref)):
        cid = cand_ref[b, j]
        vpos = cid * _CHL + lio2
        yj = xr[0] + s_col
        mys.append(jnp.where(vpos < _V, yj, _TFILL))
        fids.append(kio2 + vpos)
    my = jnp.concatenate(mys, axis=1)
    fidx = jnp.concatenate(fids, axis=1)
    for r in range(_BEAM):
        m = jnp.max(jnp.max(my, axis=1, keepdims=True), axis=0, keepdims=True)
        c1 = jnp.min(jnp.where(my == m, fidx, _BIG), axis=1, keepdims=True)
        sel = jnp.min(c1, axis=0, keepdims=True)
        vals_ref[0, :, r:r + 1] = m
        idx_ref[0, :, r:r + 1] = sel
        my = jnp.where(fidx == sel, _TFILL, my)


def _topk_body(score_ref, alive_ref, vals_ref, idx_ref):
    b = pl.program_id(0)
    x = score_ref[0]
    ridx = jax.lax.broadcasted_iota(jnp.int32, (_SUB, _LANES), 0)
    cidx = jax.lax.broadcasted_iota(jnp.int32, (_SUB, _LANES), 1)
    idx = ridx * _LANES + cidx
    s0 = alive_ref[b, 0]
    s1 = alive_ref[b, 1]
    s2 = alive_ref[b, 2]
    s3 = alive_ref[b, 3]
    add = jnp.where(idx < _V, s0, jnp.where(idx < 2 * _V, s1,
                    jnp.where(idx < 3 * _V, s2, s3)))
    y = x + add
    big = jnp.int32(2 ** 30)
    for r in range(_BEAM):
        m = jnp.max(y)
        sel = jnp.min(jnp.where(y == m, idx, big))
        vals_ref[0, 0, r] = m
        idx_ref[0, 0, r] = sel
        y = jnp.where(idx == sel, _NEG_INF, y)


def _finish_body(vals_ref, idx_ref, hyp_ref, ts_ref, as_ref, fm_ref, tok_ref, hyp_out_ref):
    top_scores = vals_ref[:, 0, :]
    index = idx_ref[:, 0, :]
    tokens = index % _V
    origin = index // _V
    hyp = hyp_ref[...]
    lane = jax.lax.broadcasted_iota(jnp.int32, (_B, _BEAM * _CUR), 1)
    grp = lane // _CUR
    zero32 = jnp.zeros((_B, _BEAM * _CUR), jnp.int32)
    origin_e = zero32
    for j in range(_BEAM):
        origin_e = jnp.where(grp == j, origin[:, j:j + 1], origin_e)
    cand = zero32
    for k in range(_BEAM):
        tile_k = jnp.concatenate([hyp[:, k * _CUR:(k + 1) * _CUR]] * _BEAM, axis=1)
        cand = jnp.where(origin_e == k, tile_k, cand)
    flags = (tokens == _EOS).astype(jnp.float32)
    alive_masked = top_scores + flags * _NEG_INF
    finish_masked = top_scores + (1.0 - flags) * _NEG_INF
    iota4 = jax.lax.broadcasted_iota(jnp.int32, (_B, _BEAM), 1)
    am = alive_masked
    new_scores = []
    new_idx = []
    for r in range(_BEAM):
        m = jnp.max(am, axis=1, keepdims=True)
        sel = jnp.min(jnp.where(am == m, iota4, _BEAM), axis=1, keepdims=True)
        new_scores.append(m)
        new_idx.append(sel)
        am = jnp.where(iota4 == sel, _NEG_INF, am)
    alive_scores_new = jnp.concatenate(new_scores, axis=1)
    alive_idx = jnp.concatenate(new_idx, axis=1)
    aidx_e = zero32
    for j in range(_BEAM):
        aidx_e = jnp.where(grp == j, alive_idx[:, j:j + 1], aidx_e)
    new_hyp = zero32
    new_tok = jnp.zeros((_B, _BEAM), jnp.int32)
    for k in range(_BEAM):
        tile_k = jnp.concatenate([cand[:, k * _CUR:(k + 1) * _CUR]] * _BEAM, axis=1)
        new_hyp = jnp.where(aidx_e == k, tile_k, new_hyp)
        new_tok = jnp.where(alive_idx == k, tokens[:, k:k + 1], new_tok)
    ts_ref[...] = top_scores
    as_ref[...] = alive_scores_new
    fm_ref[...] = finish_masked
    tok_ref[...] = tokens
    hyp_out_ref[...] = jnp.concatenate(
        [jnp.concatenate([new_hyp[:, j * _CUR:(j + 1) * _CUR],
                          new_tok[:, j:j + 1]], axis=1)
         for j in range(_BEAM)], axis=1)


def kernel(out, alive_scores, alive_hypotheses):
    cmax = pl.pallas_call(
        _scan_body,
        grid=(_B // _G,),
        in_specs=[
            pl.BlockSpec((_G, _BEAM, _V), lambda b: (b, 0, 0)),
            pl.BlockSpec(memory_space=pltpu.SMEM),
        ],
        out_specs=pl.BlockSpec((1, _G, _NCHL), lambda b: (b, 0, 0)),
        out_shape=jax.ShapeDtypeStruct((_B // _G, _G, _NCHL), jnp.float32),
    )(out, alive_scores)

    cand = pl.pallas_call(
        _chunksel_body,
        out_shape=jax.ShapeDtypeStruct((_B, _BEAM), jnp.int32),
    )(cmax.reshape(_B, _NCHL))

    vals, idx = pl.pallas_call(
        _gx_body,
        grid_spec=pltpu.PrefetchScalarGridSpec(
            num_scalar_prefetch=1,
            grid=(_B,),
            in_specs=[
                pl.BlockSpec((1, _BEAM, _CHL),
                             lambda b, cand_pref, j=j: (b, 0, cand_pref[b, j]))
                for j in range(4)
            ] + [pl.BlockSpec(memory_space=pltpu.SMEM)],
            out_specs=[
                pl.BlockSpec((1, 1, _BEAM), lambda b, cand_pref: (b, 0, 0)),
                pl.BlockSpec((1, 1, _BEAM), lambda b, cand_pref: (b, 0, 0)),
            ],
        ),
        out_shape=[
            jax.ShapeDtypeStruct((_B, 1, _BEAM), jnp.float32),
            jax.ShapeDtypeStruct((_B, 1, _BEAM), jnp.int32),
        ],
    )(cand, out, out, out, out, alive_scores)

    ts, asn, fm, tok, hyp_new = pl.pallas_call(
        _finish_body,
        out_shape=[
            jax.ShapeDtypeStruct((_B, _BEAM), jnp.float32),
            jax.ShapeDtypeStruct((_B, _BEAM), jnp.float32),
            jax.ShapeDtypeStruct((_B, _BEAM), jnp.float32),
            jax.ShapeDtypeStruct((_B, _BEAM), jnp.int32),
            jax.ShapeDtypeStruct((_B, _BEAM * (_CUR + 1)), jnp.int32),
        ],
    )(vals, idx, alive_hypotheses.reshape(_B, _BEAM * _CUR))
    return (ts, asn, fm, tok, hyp_new.reshape(_B * _BEAM, _CUR + 1))


# speedup vs baseline: 2.2115x; 1.5597x over previous
import functools

import jax
import jax.numpy as jnp
from jax import lax
from jax.experimental import pallas as pl
from jax.experimental.pallas import tpu as pltpu
from jax.experimental.pallas import tpu_sc as plsc

_B = 64
_BEAM = 4
_V = 100000
_CUR = 8
_EOS = 2
_NEG_INF = -1e20
_ROW = _BEAM * _V
_LANES = 128
_SUB = _ROW // _LANES

_NW_WORKERS = 32
_BPW = _B // _NW_WORKERS
_CHUNK = 20000
_NCHUNK = _BPW * _ROW // _CHUNK
_WIN = 2000
_NWIN = _ROW // _WIN
_FILL = -1.0e30
_BIGI = 2 ** 30
_CBUF = 512


def _sc_topk_body(scores_hbm, alive_hbm, vals_hbm, idx_hbm,
                  buf0, buf1, mw, rbuf, cval, cidx, asv, outv, outi,
                  sem0, sem1):
    wid = lax.axis_index("s") * 2 + lax.axis_index("c")
    base = wid * (_BPW * _ROW)

    pltpu.sync_copy(alive_hbm.at[pl.ds(wid * (_BPW * _BEAM), _BPW * _BEAM)],
                    asv.at[pl.ds(0, _BPW * _BEAM)])
    av = asv[...]

    def _chunk_src(c):
        return scores_hbm.at[pl.ds(base + c * _CHUNK, _CHUNK)]

    def _compute_chunk(buf, chunk_i):
        for j in range(_CHUNK // _WIN):
            def inner(t, accs):
                o = j * _WIN + t * 80
                return tuple(
                    jnp.maximum(a, buf[pl.ds(o + 16 * q, 16)])
                    for q, a in enumerate(accs))
            init = tuple(jnp.full((16,), _FILL, jnp.float32) for _ in range(5))
            a0, a1, a2, a3, a4 = lax.fori_loop(0, _WIN // 80, inner, init)
            acc = jnp.maximum(jnp.maximum(jnp.maximum(a0, a1),
                                          jnp.maximum(a2, a3)), a4)
            mw[pl.ds((chunk_i * (_CHUNK // _WIN) + j) * 16, 16)] = acc

    pltpu.async_copy(_chunk_src(0), buf0, sem0)

    def outer(i, carry):
        pltpu.async_copy(_chunk_src(2 * i + 1), buf1, sem1)
        pltpu.make_async_copy(_chunk_src(0), buf0, sem0).wait()
        _compute_chunk(buf0, 2 * i)

        @pl.when(i < _NCHUNK // 2 - 1)
        def _():
            pltpu.async_copy(_chunk_src(2 * i + 2), buf0, sem0)

        pltpu.make_async_copy(_chunk_src(1), buf1, sem1).wait()
        _compute_chunk(buf1, 2 * i + 1)
        return carry

    lax.fori_loop(0, _NCHUNK // 2, outer, jnp.int32(0))

    lane = lax.iota(jnp.int32, 16)
    ovec = jnp.full((16,), _FILL, jnp.float32)
    oivec = jnp.full((16,), 0, jnp.int32)

    for lb in range(_BPW):
        mwbase = lb * _NWIN * 16
        s0 = av[lb * _BEAM + 0]
        s1 = av[lb * _BEAM + 1]
        s2 = av[lb * _BEAM + 2]
        s3 = av[lb * _BEAM + 3]
        wps = _V // _WIN

        def _sk(wi):
            return jnp.where(wi < wps, s0,
                             jnp.where(wi < 2 * wps, s1,
                                       jnp.where(wi < 3 * wps, s2, s3)))

        def tins(wi, carry):
            t1, t2, t3, t4 = carry
            x = mw[pl.ds(mwbase + wi * 16, 16)] + _sk(wi)
            d = jnp.minimum(t1, x)
            t1 = jnp.maximum(t1, x)
            d2 = jnp.minimum(t2, d)
            t2 = jnp.maximum(t2, d)
            d3 = jnp.minimum(t3, d2)
            t3 = jnp.maximum(t3, d2)
            t4 = jnp.maximum(t4, d3)
            return (t1, t2, t3, t4)

        fill = jnp.full((16,), _FILL, jnp.float32)
        t1, t2, t3, t4 = lax.fori_loop(0, _NWIN, tins, (fill, fill, fill, fill))

        T = jnp.float32(0)
        for r in range(4):
            mm = jnp.maximum(jnp.maximum(t1, t2), jnp.maximum(t3, t4))
            T = jnp.max(mm)
            if r < 3:
                t1 = jnp.where(t1 == T, _FILL, t1)
                t2 = jnp.where(t2 == T, _FILL, t2)
                t3 = jnp.where(t3 == T, _FILL, t3)
                t4 = jnp.where(t4 == T, _FILL, t4)

        for q in range(_CBUF // 16):
            cval[pl.ds(q * 16, 16)] = fill
            cidx[pl.ds(q * 16, 16)] = jnp.full((16,), _BIGI, jnp.int32)

        def rw(wi, off):
            a = mw[pl.ds(mwbase + wi * 16, 16)]
            sk = _sk(wi)
            wmax = jnp.max(a) + sk

            def do_rescan(off):
                pltpu.sync_copy(
                    scores_hbm.at[pl.ds(base + lb * _ROW + wi * _WIN, _WIN)],
                    rbuf)

                def rv(t, off):
                    y = rbuf[pl.ds(t * 16, 16)] + sk
                    msk = y >= T
                    iv = lane + (wi * _WIN + t * 16)
                    plsc.store_compressed(cval.at[pl.ds(off, 16)], y, mask=msk)
                    plsc.store_compressed(cidx.at[pl.ds(off, 16)], iv, mask=msk)
                    cnt = plsc.all_reduce_population_count(msk)
                    return jnp.minimum(off + jnp.max(cnt),
                                       jnp.int32(_CBUF - 16))

                return lax.fori_loop(0, _WIN // 16, rv, off)

            return lax.cond(wmax >= T, do_rescan, lambda o: o, off)

        lax.fori_loop(0, _NWIN, rw, jnp.int32(0))

        for r in range(4):
            def scan_best(q, carry):
                vb, vi = carry
                v = cval[pl.ds(q * 16, 16)]
                ix = cidx[pl.ds(q * 16, 16)]
                better = (v > vb) | ((v == vb) & (ix < vi))
                return (jnp.where(better, v, vb), jnp.where(better, ix, vi))

            vb, vi = lax.fori_loop(
                0, _CBUF // 16, scan_best,
                (fill, jnp.full((16,), _BIGI, jnp.int32)))
            m = jnp.max(vb)
            mi = jnp.min(jnp.where(vb == m, vi, _BIGI))

            def rem(q, carry):
                ix = cidx[pl.ds(q * 16, 16)]
                v = cval[pl.ds(q * 16, 16)]
                cval[pl.ds(q * 16, 16)] = jnp.where(ix == mi, _FILL, v)
                return carry

            lax.fori_loop(0, _CBUF // 16, rem, jnp.int32(0))
            pos = lb * _BEAM + r
            ovec = jnp.where(lane == pos, m, ovec)
            oivec = jnp.where(lane == pos, mi, oivec)

    outv[...] = ovec
    outi[...] = oivec
    n_out = _BPW * _BEAM
    pltpu.sync_copy(outv.at[pl.ds(0, n_out)], vals_hbm.at[pl.ds(wid * n_out, n_out)])
    pltpu.sync_copy(outi.at[pl.ds(0, n_out)], idx_hbm.at[pl.ds(wid * n_out, n_out)])


def _make_sc_topk():
    return functools.partial(
        pl.kernel,
        mesh=plsc.VectorSubcoreMesh(core_axis_name="c", subcore_axis_name="s"),
        compiler_params=pltpu.CompilerParams(needs_layout_passes=False),
        out_type=[
        jax.ShapeDtypeStruct((_B * _BEAM,), jnp.float32),
        jax.ShapeDtypeStruct((_B * _BEAM,), jnp.int32),
    ],
    scratch_types=[
        pltpu.VMEM((_CHUNK,), jnp.float32),
        pltpu.VMEM((_CHUNK,), jnp.float32),
        pltpu.VMEM((_BPW * _NWIN * 16,), jnp.float32),
        pltpu.VMEM((_WIN,), jnp.float32),
        pltpu.VMEM((_CBUF,), jnp.float32),
        pltpu.VMEM((_CBUF,), jnp.int32),
        pltpu.VMEM((16,), jnp.float32),
        pltpu.VMEM((16,), jnp.float32),
        pltpu.VMEM((16,), jnp.int32),
            pltpu.SemaphoreType.DMA,
            pltpu.SemaphoreType.DMA,
        ],
    )(_sc_topk_body)


_CHL = 2048
_NCHL = 49
_TAIL_A = 48 * _CHL
_TFILL = -3.0e38
_G = 4
_BIG = 2 ** 30


def _scan_body(x_ref, alive_ref, cmax_ref):
    i0 = pl.program_id(0) * _G
    br = jax.lax.broadcasted_iota(jnp.int32, (_BEAM, 1), 0)
    for g in range(_G):
        s_list = [alive_ref[i0 + g, k] for k in range(_BEAM)]
        s_col = jnp.where(br == 0, s_list[0],
                          jnp.where(br == 1, s_list[1],
                                    jnp.where(br == 2, s_list[2], s_list[3])))
        cms = []
        for c in range(_NCHL):
            a = c * _CHL
            if c < _NCHL - 1:
                w = x_ref[g, :, a:a + 128]
                for t in range(1, _CHL // 128):
                    w = jnp.maximum(w, x_ref[g, :, a + 128 * t:a + 128 * (t + 1)])
                cm = jnp.max(w, axis=1, keepdims=True)
            else:
                w = x_ref[g, :, a:a + 128]
                for t in range(1, (_V - _TAIL_A) // 128):
                    w = jnp.maximum(w, x_ref[g, :, a + 128 * t:a + 128 * (t + 1)])
                cm = jnp.maximum(
                    jnp.max(w, axis=1, keepdims=True),
                    jnp.max(x_ref[g, :, _TAIL_A + ((_V - _TAIL_A) // 128) * 128:_V],
                            axis=1, keepdims=True))
            cms.append(cm)
        cmat = jnp.concatenate(cms, axis=1) + s_col
        cmax_ref[0, g:g + 1, :] = jnp.max(cmat, axis=0, keepdims=True)


def _chunksel_body(cmax_ref, cand_ref):
    cm = cmax_ref[...]
    cio = jax.lax.broadcasted_iota(jnp.int32, (_B, _NCHL), 1)
    ids = []
    for r in range(_BEAM):
        m = jnp.max(cm, axis=1, keepdims=True)
        cid = jnp.min(jnp.where(cm == m, cio, _BIG), axis=1, keepdims=True)
        ids.append(cid)
        cm = jnp.where(cio == cid, _TFILL, cm)
    cand_ref[...] = jnp.concatenate(ids, axis=1)


def _gx_body(cand_ref, *refs):
    xrefs = refs[:16]
    alive_ref = refs[16]
    vals_ref, idx_ref = refs[17], refs[18]
    i0 = pl.program_id(0) * _G
    kio2 = jax.lax.broadcasted_iota(jnp.int32, (_BEAM, _CHL), 0) * _V
    lio2 = jax.lax.broadcasted_iota(jnp.int32, (_BEAM, _CHL), 1)
    br = jax.lax.broadcasted_iota(jnp.int32, (_BEAM, 1), 0)
    for g in range(_G):
        b = i0 + g
        s_list = [alive_ref[b, k] for k in range(_BEAM)]
        s_col = jnp.where(br == 0, s_list[0],
                          jnp.where(br == 1, s_list[1],
                                    jnp.where(br == 2, s_list[2], s_list[3])))
        mys = []
        fids = []
        for j in range(4):
            xr = xrefs[g * 4 + j]
            cid = cand_ref[b, j]
            vpos = cid * _CHL + lio2
            yj = xr[0] + s_col
            mys.append(jnp.where(vpos < _V, yj, _TFILL))
            fids.append(kio2 + vpos)
        my = jnp.concatenate(mys, axis=1)
        fidx = jnp.concatenate(fids, axis=1)
        for r in range(_BEAM):
            m = jnp.max(jnp.max(my, axis=1, keepdims=True), axis=0,
                        keepdims=True)
            c1 = jnp.min(jnp.where(my == m, fidx, _BIG), axis=1, keepdims=True)
            sel = jnp.min(c1, axis=0, keepdims=True)
            vals_ref[g, :, r:r + 1] = m
            idx_ref[g, :, r:r + 1] = sel
            my = jnp.where(fidx == sel, _TFILL, my)


def _topk_body(score_ref, alive_ref, vals_ref, idx_ref):
    b = pl.program_id(0)
    x = score_ref[0]
    ridx = jax.lax.broadcasted_iota(jnp.int32, (_SUB, _LANES), 0)
    cidx = jax.lax.broadcasted_iota(jnp.int32, (_SUB, _LANES), 1)
    idx = ridx * _LANES + cidx
    s0 = alive_ref[b, 0]
    s1 = alive_ref[b, 1]
    s2 = alive_ref[b, 2]
    s3 = alive_ref[b, 3]
    add = jnp.where(idx < _V, s0, jnp.where(idx < 2 * _V, s1,
                    jnp.where(idx < 3 * _V, s2, s3)))
    y = x + add
    big = jnp.int32(2 ** 30)
    for r in range(_BEAM):
        m = jnp.max(y)
        sel = jnp.min(jnp.where(y == m, idx, big))
        vals_ref[0, 0, r] = m
        idx_ref[0, 0, r] = sel
        y = jnp.where(idx == sel, _NEG_INF, y)


def _finish_body(vals_ref, idx_ref, hyp_ref, ts_ref, as_ref, fm_ref, tok_ref, hyp_out_ref):
    top_scores = vals_ref[:, 0, :]
    index = idx_ref[:, 0, :]
    tokens = index % _V
    origin = index // _V
    hyp = hyp_ref[...]
    lane = jax.lax.broadcasted_iota(jnp.int32, (_B, _BEAM * _CUR), 1)
    grp = lane // _CUR
    zero32 = jnp.zeros((_B, _BEAM * _CUR), jnp.int32)
    origin_e = zero32
    for j in range(_BEAM):
        origin_e = jnp.where(grp == j, origin[:, j:j + 1], origin_e)
    cand = zero32
    for k in range(_BEAM):
        tile_k = jnp.concatenate([hyp[:, k * _CUR:(k + 1) * _CUR]] * _BEAM, axis=1)
        cand = jnp.where(origin_e == k, tile_k, cand)
    flags = (tokens == _EOS).astype(jnp.float32)
    alive_masked = top_scores + flags * _NEG_INF
    finish_masked = top_scores + (1.0 - flags) * _NEG_INF
    iota4 = jax.lax.broadcasted_iota(jnp.int32, (_B, _BEAM), 1)
    am = alive_masked
    new_scores = []
    new_idx = []
    for r in range(_BEAM):
        m = jnp.max(am, axis=1, keepdims=True)
        sel = jnp.min(jnp.where(am == m, iota4, _BEAM), axis=1, keepdims=True)
        new_scores.append(m)
        new_idx.append(sel)
        am = jnp.where(iota4 == sel, _NEG_INF, am)
    alive_scores_new = jnp.concatenate(new_scores, axis=1)
    alive_idx = jnp.concatenate(new_idx, axis=1)
    aidx_e = zero32
    for j in range(_BEAM):
        aidx_e = jnp.where(grp == j, alive_idx[:, j:j + 1], aidx_e)
    new_hyp = zero32
    new_tok = jnp.zeros((_B, _BEAM), jnp.int32)
    for k in range(_BEAM):
        tile_k = jnp.concatenate([cand[:, k * _CUR:(k + 1) * _CUR]] * _BEAM, axis=1)
        new_hyp = jnp.where(aidx_e == k, tile_k, new_hyp)
        new_tok = jnp.where(alive_idx == k, tokens[:, k:k + 1], new_tok)
    ts_ref[...] = top_scores
    as_ref[...] = alive_scores_new
    fm_ref[...] = finish_masked
    tok_ref[...] = tokens
    hyp_out_ref[...] = jnp.concatenate(
        [jnp.concatenate([new_hyp[:, j * _CUR:(j + 1) * _CUR],
                          new_tok[:, j:j + 1]], axis=1)
         for j in range(_BEAM)], axis=1)


def kernel(out, alive_scores, alive_hypotheses):
    cmax = pl.pallas_call(
        _scan_body,
        grid=(_B // _G,),
        in_specs=[
            pl.BlockSpec((_G, _BEAM, _V), lambda b: (b, 0, 0)),
            pl.BlockSpec(memory_space=pltpu.SMEM),
        ],
        out_specs=pl.BlockSpec((1, _G, _NCHL), lambda b: (b, 0, 0)),
        out_shape=jax.ShapeDtypeStruct((_B // _G, _G, _NCHL), jnp.float32),
    )(out, alive_scores)

    cand = pl.pallas_call(
        _chunksel_body,
        out_shape=jax.ShapeDtypeStruct((_B, _BEAM), jnp.int32),
    )(cmax.reshape(_B, _NCHL))

    vals, idx = pl.pallas_call(
        _gx_body,
        grid_spec=pltpu.PrefetchScalarGridSpec(
            num_scalar_prefetch=1,
            grid=(_B // _G,),
            in_specs=[
                pl.BlockSpec(
                    (1, _BEAM, _CHL),
                    lambda i, cand_pref, g=g, j=j: (
                        i * _G + g, 0, cand_pref[i * _G + g, j]))
                for g in range(_G) for j in range(4)
            ] + [pl.BlockSpec(memory_space=pltpu.SMEM)],
            out_specs=[
                pl.BlockSpec((_G, 1, _BEAM), lambda i, cand_pref: (i, 0, 0)),
                pl.BlockSpec((_G, 1, _BEAM), lambda i, cand_pref: (i, 0, 0)),
            ],
        ),
        out_shape=[
            jax.ShapeDtypeStruct((_B, 1, _BEAM), jnp.float32),
            jax.ShapeDtypeStruct((_B, 1, _BEAM), jnp.int32),
        ],
    )(cand, *([out] * 16), alive_scores)

    ts, asn, fm, tok, hyp_new = pl.pallas_call(
        _finish_body,
        out_shape=[
            jax.ShapeDtypeStruct((_B, _BEAM), jnp.float32),
            jax.ShapeDtypeStruct((_B, _BEAM), jnp.float32),
            jax.ShapeDtypeStruct((_B, _BEAM), jnp.float32),
            jax.ShapeDtypeStruct((_B, _BEAM), jnp.int32),
            jax.ShapeDtypeStruct((_B, _BEAM * (_CUR + 1)), jnp.int32),
        ],
    )(vals, idx, alive_hypotheses.reshape(_B, _BEAM * _CUR))
    return (ts, asn, fm, tok, hyp_new.reshape(_B * _BEAM, _CUR + 1))


# speedup vs baseline: 2.2954x; 1.0380x over previous
import functools

import jax
import jax.numpy as jnp
from jax import lax
from jax.experimental import pallas as pl
from jax.experimental.pallas import tpu as pltpu
from jax.experimental.pallas import tpu_sc as plsc

_B = 64
_BEAM = 4
_V = 100000
_CUR = 8
_EOS = 2
_NEG_INF = -1e20
_ROW = _BEAM * _V
_LANES = 128
_SUB = _ROW // _LANES

_NW_WORKERS = 32
_BPW = _B // _NW_WORKERS
_CHUNK = 20000
_NCHUNK = _BPW * _ROW // _CHUNK
_WIN = 2000
_NWIN = _ROW // _WIN
_FILL = -1.0e30
_BIGI = 2 ** 30
_CBUF = 512


def _sc_topk_body(scores_hbm, alive_hbm, vals_hbm, idx_hbm,
                  buf0, buf1, mw, rbuf, cval, cidx, asv, outv, outi,
                  sem0, sem1):
    wid = lax.axis_index("s") * 2 + lax.axis_index("c")
    base = wid * (_BPW * _ROW)

    pltpu.sync_copy(alive_hbm.at[pl.ds(wid * (_BPW * _BEAM), _BPW * _BEAM)],
                    asv.at[pl.ds(0, _BPW * _BEAM)])
    av = asv[...]

    def _chunk_src(c):
        return scores_hbm.at[pl.ds(base + c * _CHUNK, _CHUNK)]

    def _compute_chunk(buf, chunk_i):
        for j in range(_CHUNK // _WIN):
            def inner(t, accs):
                o = j * _WIN + t * 80
                return tuple(
                    jnp.maximum(a, buf[pl.ds(o + 16 * q, 16)])
                    for q, a in enumerate(accs))
            init = tuple(jnp.full((16,), _FILL, jnp.float32) for _ in range(5))
            a0, a1, a2, a3, a4 = lax.fori_loop(0, _WIN // 80, inner, init)
            acc = jnp.maximum(jnp.maximum(jnp.maximum(a0, a1),
                                          jnp.maximum(a2, a3)), a4)
            mw[pl.ds((chunk_i * (_CHUNK // _WIN) + j) * 16, 16)] = acc

    pltpu.async_copy(_chunk_src(0), buf0, sem0)

    def outer(i, carry):
        pltpu.async_copy(_chunk_src(2 * i + 1), buf1, sem1)
        pltpu.make_async_copy(_chunk_src(0), buf0, sem0).wait()
        _compute_chunk(buf0, 2 * i)

        @pl.when(i < _NCHUNK // 2 - 1)
        def _():
            pltpu.async_copy(_chunk_src(2 * i + 2), buf0, sem0)

        pltpu.make_async_copy(_chunk_src(1), buf1, sem1).wait()
        _compute_chunk(buf1, 2 * i + 1)
        return carry

    lax.fori_loop(0, _NCHUNK // 2, outer, jnp.int32(0))

    lane = lax.iota(jnp.int32, 16)
    ovec = jnp.full((16,), _FILL, jnp.float32)
    oivec = jnp.full((16,), 0, jnp.int32)

    for lb in range(_BPW):
        mwbase = lb * _NWIN * 16
        s0 = av[lb * _BEAM + 0]
        s1 = av[lb * _BEAM + 1]
        s2 = av[lb * _BEAM + 2]
        s3 = av[lb * _BEAM + 3]
        wps = _V // _WIN

        def _sk(wi):
            return jnp.where(wi < wps, s0,
                             jnp.where(wi < 2 * wps, s1,
                                       jnp.where(wi < 3 * wps, s2, s3)))

        def tins(wi, carry):
            t1, t2, t3, t4 = carry
            x = mw[pl.ds(mwbase + wi * 16, 16)] + _sk(wi)
            d = jnp.minimum(t1, x)
            t1 = jnp.maximum(t1, x)
            d2 = jnp.minimum(t2, d)
            t2 = jnp.maximum(t2, d)
            d3 = jnp.minimum(t3, d2)
            t3 = jnp.maximum(t3, d2)
            t4 = jnp.maximum(t4, d3)
            return (t1, t2, t3, t4)

        fill = jnp.full((16,), _FILL, jnp.float32)
        t1, t2, t3, t4 = lax.fori_loop(0, _NWIN, tins, (fill, fill, fill, fill))

        T = jnp.float32(0)
        for r in range(4):
            mm = jnp.maximum(jnp.maximum(t1, t2), jnp.maximum(t3, t4))
            T = jnp.max(mm)
            if r < 3:
                t1 = jnp.where(t1 == T, _FILL, t1)
                t2 = jnp.where(t2 == T, _FILL, t2)
                t3 = jnp.where(t3 == T, _FILL, t3)
                t4 = jnp.where(t4 == T, _FILL, t4)

        for q in range(_CBUF // 16):
            cval[pl.ds(q * 16, 16)] = fill
            cidx[pl.ds(q * 16, 16)] = jnp.full((16,), _BIGI, jnp.int32)

        def rw(wi, off):
            a = mw[pl.ds(mwbase + wi * 16, 16)]
            sk = _sk(wi)
            wmax = jnp.max(a) + sk

            def do_rescan(off):
                pltpu.sync_copy(
                    scores_hbm.at[pl.ds(base + lb * _ROW + wi * _WIN, _WIN)],
                    rbuf)

                def rv(t, off):
                    y = rbuf[pl.ds(t * 16, 16)] + sk
                    msk = y >= T
                    iv = lane + (wi * _WIN + t * 16)
                    plsc.store_compressed(cval.at[pl.ds(off, 16)], y, mask=msk)
                    plsc.store_compressed(cidx.at[pl.ds(off, 16)], iv, mask=msk)
                    cnt = plsc.all_reduce_population_count(msk)
                    return jnp.minimum(off + jnp.max(cnt),
                                       jnp.int32(_CBUF - 16))

                return lax.fori_loop(0, _WIN // 16, rv, off)

            return lax.cond(wmax >= T, do_rescan, lambda o: o, off)

        lax.fori_loop(0, _NWIN, rw, jnp.int32(0))

        for r in range(4):
            def scan_best(q, carry):
                vb, vi = carry
                v = cval[pl.ds(q * 16, 16)]
                ix = cidx[pl.ds(q * 16, 16)]
                better = (v > vb) | ((v == vb) & (ix < vi))
                return (jnp.where(better, v, vb), jnp.where(better, ix, vi))

            vb, vi = lax.fori_loop(
                0, _CBUF // 16, scan_best,
                (fill, jnp.full((16,), _BIGI, jnp.int32)))
            m = jnp.max(vb)
            mi = jnp.min(jnp.where(vb == m, vi, _BIGI))

            def rem(q, carry):
                ix = cidx[pl.ds(q * 16, 16)]
                v = cval[pl.ds(q * 16, 16)]
                cval[pl.ds(q * 16, 16)] = jnp.where(ix == mi, _FILL, v)
                return carry

            lax.fori_loop(0, _CBUF // 16, rem, jnp.int32(0))
            pos = lb * _BEAM + r
            ovec = jnp.where(lane == pos, m, ovec)
            oivec = jnp.where(lane == pos, mi, oivec)

    outv[...] = ovec
    outi[...] = oivec
    n_out = _BPW * _BEAM
    pltpu.sync_copy(outv.at[pl.ds(0, n_out)], vals_hbm.at[pl.ds(wid * n_out, n_out)])
    pltpu.sync_copy(outi.at[pl.ds(0, n_out)], idx_hbm.at[pl.ds(wid * n_out, n_out)])


def _make_sc_topk():
    return functools.partial(
        pl.kernel,
        mesh=plsc.VectorSubcoreMesh(core_axis_name="c", subcore_axis_name="s"),
        compiler_params=pltpu.CompilerParams(needs_layout_passes=False),
        out_type=[
        jax.ShapeDtypeStruct((_B * _BEAM,), jnp.float32),
        jax.ShapeDtypeStruct((_B * _BEAM,), jnp.int32),
    ],
    scratch_types=[
        pltpu.VMEM((_CHUNK,), jnp.float32),
        pltpu.VMEM((_CHUNK,), jnp.float32),
        pltpu.VMEM((_BPW * _NWIN * 16,), jnp.float32),
        pltpu.VMEM((_WIN,), jnp.float32),
        pltpu.VMEM((_CBUF,), jnp.float32),
        pltpu.VMEM((_CBUF,), jnp.int32),
        pltpu.VMEM((16,), jnp.float32),
        pltpu.VMEM((16,), jnp.float32),
        pltpu.VMEM((16,), jnp.int32),
            pltpu.SemaphoreType.DMA,
            pltpu.SemaphoreType.DMA,
        ],
    )(_sc_topk_body)


_CHL = 2048
_NCHL = 49
_TAIL_A = 48 * _CHL
_TFILL = -3.0e38
_G = 4
_GX = 8
_BIG = 2 ** 30


def _scan_body(x_ref, alive_ref, cmax_ref):
    i0 = pl.program_id(0) * _G
    br = jax.lax.broadcasted_iota(jnp.int32, (_BEAM, 1), 0)
    for g in range(_G):
        s_list = [alive_ref[i0 + g, k] for k in range(_BEAM)]
        s_col = jnp.where(br == 0, s_list[0],
                          jnp.where(br == 1, s_list[1],
                                    jnp.where(br == 2, s_list[2], s_list[3])))
        cms = []
        for c in range(_NCHL):
            a = c * _CHL
            if c < _NCHL - 1:
                w = x_ref[g, :, a:a + 128]
                for t in range(1, _CHL // 128):
                    w = jnp.maximum(w, x_ref[g, :, a + 128 * t:a + 128 * (t + 1)])
                cm = jnp.max(w, axis=1, keepdims=True)
            else:
                w = x_ref[g, :, a:a + 128]
                for t in range(1, (_V - _TAIL_A) // 128):
                    w = jnp.maximum(w, x_ref[g, :, a + 128 * t:a + 128 * (t + 1)])
                cm = jnp.maximum(
                    jnp.max(w, axis=1, keepdims=True),
                    jnp.max(x_ref[g, :, _TAIL_A + ((_V - _TAIL_A) // 128) * 128:_V],
                            axis=1, keepdims=True))
            cms.append(cm)
        cmat = jnp.concatenate(cms, axis=1) + s_col
        cmax_ref[0, g:g + 1, :] = jnp.max(cmat, axis=0, keepdims=True)


def _chunksel_body(cmax_ref, cand_ref):
    cm = cmax_ref[...]
    cio = jax.lax.broadcasted_iota(jnp.int32, (_B, _NCHL), 1)
    ids = []
    for r in range(_BEAM):
        m = jnp.max(cm, axis=1, keepdims=True)
        cid = jnp.min(jnp.where(cm == m, cio, _BIG), axis=1, keepdims=True)
        ids.append(cid)
        cm = jnp.where(cio == cid, _TFILL, cm)
    cand_ref[...] = jnp.concatenate(ids, axis=1)


def _gx_body(cand_ref, *refs):
    xrefs = refs[:4 * _GX]
    alive_ref = refs[4 * _GX]
    vals_ref, idx_ref = refs[4 * _GX + 1], refs[4 * _GX + 2]
    i0 = pl.program_id(0) * _GX
    kio2 = jax.lax.broadcasted_iota(jnp.int32, (_BEAM, _CHL), 0) * _V
    lio2 = jax.lax.broadcasted_iota(jnp.int32, (_BEAM, _CHL), 1)
    br = jax.lax.broadcasted_iota(jnp.int32, (_BEAM, 1), 0)
    for g in range(_GX):
        b = i0 + g
        s_list = [alive_ref[b, k] for k in range(_BEAM)]
        s_col = jnp.where(br == 0, s_list[0],
                          jnp.where(br == 1, s_list[1],
                                    jnp.where(br == 2, s_list[2], s_list[3])))
        mys = []
        fids = []
        for j in range(4):
            xr = xrefs[g * 4 + j]
            cid = cand_ref[b, j]
            vpos = cid * _CHL + lio2
            yj = xr[0] + s_col
            mys.append(jnp.where(vpos < _V, yj, _TFILL))
            fids.append(kio2 + vpos)
        my = jnp.concatenate(mys, axis=1)
        fidx = jnp.concatenate(fids, axis=1)
        for r in range(_BEAM):
            m = jnp.max(jnp.max(my, axis=1, keepdims=True), axis=0,
                        keepdims=True)
            c1 = jnp.min(jnp.where(my == m, fidx, _BIG), axis=1, keepdims=True)
            sel = jnp.min(c1, axis=0, keepdims=True)
            vals_ref[g, :, r:r + 1] = m
            idx_ref[g, :, r:r + 1] = sel
            my = jnp.where(fidx == sel, _TFILL, my)


def _topk_body(score_ref, alive_ref, vals_ref, idx_ref):
    b = pl.program_id(0)
    x = score_ref[0]
    ridx = jax.lax.broadcasted_iota(jnp.int32, (_SUB, _LANES), 0)
    cidx = jax.lax.broadcasted_iota(jnp.int32, (_SUB, _LANES), 1)
    idx = ridx * _LANES + cidx
    s0 = alive_ref[b, 0]
    s1 = alive_ref[b, 1]
    s2 = alive_ref[b, 2]
    s3 = alive_ref[b, 3]
    add = jnp.where(idx < _V, s0, jnp.where(idx < 2 * _V, s1,
                    jnp.where(idx < 3 * _V, s2, s3)))
    y = x + add
    big = jnp.int32(2 ** 30)
    for r in range(_BEAM):
        m = jnp.max(y)
        sel = jnp.min(jnp.where(y == m, idx, big))
        vals_ref[0, 0, r] = m
        idx_ref[0, 0, r] = sel
        y = jnp.where(idx == sel, _NEG_INF, y)


def _finish_body(vals_ref, idx_ref, hyp_ref, ts_ref, as_ref, fm_ref, tok_ref, hyp_out_ref):
    top_scores = vals_ref[:, 0, :]
    index = idx_ref[:, 0, :]
    tokens = index % _V
    origin = index // _V
    hyp = hyp_ref[...]
    lane = jax.lax.broadcasted_iota(jnp.int32, (_B, _BEAM * _CUR), 1)
    grp = lane // _CUR
    zero32 = jnp.zeros((_B, _BEAM * _CUR), jnp.int32)
    origin_e = zero32
    for j in range(_BEAM):
        origin_e = jnp.where(grp == j, origin[:, j:j + 1], origin_e)
    cand = zero32
    for k in range(_BEAM):
        tile_k = jnp.concatenate([hyp[:, k * _CUR:(k + 1) * _CUR]] * _BEAM, axis=1)
        cand = jnp.where(origin_e == k, tile_k, cand)
    flags = (tokens == _EOS).astype(jnp.float32)
    alive_masked = top_scores + flags * _NEG_INF
    finish_masked = top_scores + (1.0 - flags) * _NEG_INF
    iota4 = jax.lax.broadcasted_iota(jnp.int32, (_B, _BEAM), 1)
    am = alive_masked
    new_scores = []
    new_idx = []
    for r in range(_BEAM):
        m = jnp.max(am, axis=1, keepdims=True)
        sel = jnp.min(jnp.where(am == m, iota4, _BEAM), axis=1, keepdims=True)
        new_scores.append(m)
        new_idx.append(sel)
        am = jnp.where(iota4 == sel, _NEG_INF, am)
    alive_scores_new = jnp.concatenate(new_scores, axis=1)
    alive_idx = jnp.concatenate(new_idx, axis=1)
    aidx_e = zero32
    for j in range(_BEAM):
        aidx_e = jnp.where(grp == j, alive_idx[:, j:j + 1], aidx_e)
    new_hyp = zero32
    new_tok = jnp.zeros((_B, _BEAM), jnp.int32)
    for k in range(_BEAM):
        tile_k = jnp.concatenate([cand[:, k * _CUR:(k + 1) * _CUR]] * _BEAM, axis=1)
        new_hyp = jnp.where(aidx_e == k, tile_k, new_hyp)
        new_tok = jnp.where(alive_idx == k, tokens[:, k:k + 1], new_tok)
    ts_ref[...] = top_scores
    as_ref[...] = alive_scores_new
    fm_ref[...] = finish_masked
    tok_ref[...] = tokens
    hyp_out_ref[...] = jnp.concatenate(
        [jnp.concatenate([new_hyp[:, j * _CUR:(j + 1) * _CUR],
                          new_tok[:, j:j + 1]], axis=1)
         for j in range(_BEAM)], axis=1)


def kernel(out, alive_scores, alive_hypotheses):
    cmax = pl.pallas_call(
        _scan_body,
        grid=(_B // _G,),
        in_specs=[
            pl.BlockSpec((_G, _BEAM, _V), lambda b: (b, 0, 0)),
            pl.BlockSpec(memory_space=pltpu.SMEM),
        ],
        out_specs=pl.BlockSpec((1, _G, _NCHL), lambda b: (b, 0, 0)),
        out_shape=jax.ShapeDtypeStruct((_B // _G, _G, _NCHL), jnp.float32),
    )(out, alive_scores)

    cand = pl.pallas_call(
        _chunksel_body,
        out_shape=jax.ShapeDtypeStruct((_B, _BEAM), jnp.int32),
    )(cmax.reshape(_B, _NCHL))

    vals, idx = pl.pallas_call(
        _gx_body,
        grid_spec=pltpu.PrefetchScalarGridSpec(
            num_scalar_prefetch=1,
            grid=(_B // _GX,),
            in_specs=[
                pl.BlockSpec(
                    (1, _BEAM, _CHL),
                    lambda i, cand_pref, g=g, j=j: (
                        i * _GX + g, 0, cand_pref[i * _GX + g, j]))
                for g in range(_GX) for j in range(4)
            ] + [pl.BlockSpec(memory_space=pltpu.SMEM)],
            out_specs=[
                pl.BlockSpec((_GX, 1, _BEAM), lambda i, cand_pref: (i, 0, 0)),
                pl.BlockSpec((_GX, 1, _BEAM), lambda i, cand_pref: (i, 0, 0)),
            ],
        ),
        out_shape=[
            jax.ShapeDtypeStruct((_B, 1, _BEAM), jnp.float32),
            jax.ShapeDtypeStruct((_B, 1, _BEAM), jnp.int32),
        ],
    )(cand, *([out] * (4 * _GX)), alive_scores)

    ts, asn, fm, tok, hyp_new = pl.pallas_call(
        _finish_body,
        out_shape=[
            jax.ShapeDtypeStruct((_B, _BEAM), jnp.float32),
            jax.ShapeDtypeStruct((_B, _BEAM), jnp.float32),
            jax.ShapeDtypeStruct((_B, _BEAM), jnp.float32),
            jax.ShapeDtypeStruct((_B, _BEAM), jnp.int32),
            jax.ShapeDtypeStruct((_B, _BEAM * (_CUR + 1)), jnp.int32),
        ],
    )(vals, idx, alive_hypotheses.reshape(_B, _BEAM * _CUR))
    return (ts, asn, fm, tok, hyp_new.reshape(_B * _BEAM, _CUR + 1))


# speedup vs baseline: 2.7484x; 1.1974x over previous
import functools

import jax
import jax.numpy as jnp
from jax import lax
from jax.experimental import pallas as pl
from jax.experimental.pallas import tpu as pltpu
from jax.experimental.pallas import tpu_sc as plsc

_B = 64
_BEAM = 4
_V = 100000
_CUR = 8
_EOS = 2
_NEG_INF = -1e20
_ROW = _BEAM * _V
_LANES = 128
_SUB = _ROW // _LANES

_NW_WORKERS = 32
_BPW = _B // _NW_WORKERS
_CHUNK = 20000
_NCHUNK = _BPW * _ROW // _CHUNK
_WIN = 2000
_NWIN = _ROW // _WIN
_FILL = -1.0e30
_BIGI = 2 ** 30
_CBUF = 512


def _sc_topk_body(scores_hbm, alive_hbm, vals_hbm, idx_hbm,
                  buf0, buf1, mw, rbuf, cval, cidx, asv, outv, outi,
                  sem0, sem1):
    wid = lax.axis_index("s") * 2 + lax.axis_index("c")
    base = wid * (_BPW * _ROW)

    pltpu.sync_copy(alive_hbm.at[pl.ds(wid * (_BPW * _BEAM), _BPW * _BEAM)],
                    asv.at[pl.ds(0, _BPW * _BEAM)])
    av = asv[...]

    def _chunk_src(c):
        return scores_hbm.at[pl.ds(base + c * _CHUNK, _CHUNK)]

    def _compute_chunk(buf, chunk_i):
        for j in range(_CHUNK // _WIN):
            def inner(t, accs):
                o = j * _WIN + t * 80
                return tuple(
                    jnp.maximum(a, buf[pl.ds(o + 16 * q, 16)])
                    for q, a in enumerate(accs))
            init = tuple(jnp.full((16,), _FILL, jnp.float32) for _ in range(5))
            a0, a1, a2, a3, a4 = lax.fori_loop(0, _WIN // 80, inner, init)
            acc = jnp.maximum(jnp.maximum(jnp.maximum(a0, a1),
                                          jnp.maximum(a2, a3)), a4)
            mw[pl.ds((chunk_i * (_CHUNK // _WIN) + j) * 16, 16)] = acc

    pltpu.async_copy(_chunk_src(0), buf0, sem0)

    def outer(i, carry):
        pltpu.async_copy(_chunk_src(2 * i + 1), buf1, sem1)
        pltpu.make_async_copy(_chunk_src(0), buf0, sem0).wait()
        _compute_chunk(buf0, 2 * i)

        @pl.when(i < _NCHUNK // 2 - 1)
        def _():
            pltpu.async_copy(_chunk_src(2 * i + 2), buf0, sem0)

        pltpu.make_async_copy(_chunk_src(1), buf1, sem1).wait()
        _compute_chunk(buf1, 2 * i + 1)
        return carry

    lax.fori_loop(0, _NCHUNK // 2, outer, jnp.int32(0))

    lane = lax.iota(jnp.int32, 16)
    ovec = jnp.full((16,), _FILL, jnp.float32)
    oivec = jnp.full((16,), 0, jnp.int32)

    for lb in range(_BPW):
        mwbase = lb * _NWIN * 16
        s0 = av[lb * _BEAM + 0]
        s1 = av[lb * _BEAM + 1]
        s2 = av[lb * _BEAM + 2]
        s3 = av[lb * _BEAM + 3]
        wps = _V // _WIN

        def _sk(wi):
            return jnp.where(wi < wps, s0,
                             jnp.where(wi < 2 * wps, s1,
                                       jnp.where(wi < 3 * wps, s2, s3)))

        def tins(wi, carry):
            t1, t2, t3, t4 = carry
            x = mw[pl.ds(mwbase + wi * 16, 16)] + _sk(wi)
            d = jnp.minimum(t1, x)
            t1 = jnp.maximum(t1, x)
            d2 = jnp.minimum(t2, d)
            t2 = jnp.maximum(t2, d)
            d3 = jnp.minimum(t3, d2)
            t3 = jnp.maximum(t3, d2)
            t4 = jnp.maximum(t4, d3)
            return (t1, t2, t3, t4)

        fill = jnp.full((16,), _FILL, jnp.float32)
        t1, t2, t3, t4 = lax.fori_loop(0, _NWIN, tins, (fill, fill, fill, fill))

        T = jnp.float32(0)
        for r in range(4):
            mm = jnp.maximum(jnp.maximum(t1, t2), jnp.maximum(t3, t4))
            T = jnp.max(mm)
            if r < 3:
                t1 = jnp.where(t1 == T, _FILL, t1)
                t2 = jnp.where(t2 == T, _FILL, t2)
                t3 = jnp.where(t3 == T, _FILL, t3)
                t4 = jnp.where(t4 == T, _FILL, t4)

        for q in range(_CBUF // 16):
            cval[pl.ds(q * 16, 16)] = fill
            cidx[pl.ds(q * 16, 16)] = jnp.full((16,), _BIGI, jnp.int32)

        def rw(wi, off):
            a = mw[pl.ds(mwbase + wi * 16, 16)]
            sk = _sk(wi)
            wmax = jnp.max(a) + sk

            def do_rescan(off):
                pltpu.sync_copy(
                    scores_hbm.at[pl.ds(base + lb * _ROW + wi * _WIN, _WIN)],
                    rbuf)

                def rv(t, off):
                    y = rbuf[pl.ds(t * 16, 16)] + sk
                    msk = y >= T
                    iv = lane + (wi * _WIN + t * 16)
                    plsc.store_compressed(cval.at[pl.ds(off, 16)], y, mask=msk)
                    plsc.store_compressed(cidx.at[pl.ds(off, 16)], iv, mask=msk)
                    cnt = plsc.all_reduce_population_count(msk)
                    return jnp.minimum(off + jnp.max(cnt),
                                       jnp.int32(_CBUF - 16))

                return lax.fori_loop(0, _WIN // 16, rv, off)

            return lax.cond(wmax >= T, do_rescan, lambda o: o, off)

        lax.fori_loop(0, _NWIN, rw, jnp.int32(0))

        for r in range(4):
            def scan_best(q, carry):
                vb, vi = carry
                v = cval[pl.ds(q * 16, 16)]
                ix = cidx[pl.ds(q * 16, 16)]
                better = (v > vb) | ((v == vb) & (ix < vi))
                return (jnp.where(better, v, vb), jnp.where(better, ix, vi))

            vb, vi = lax.fori_loop(
                0, _CBUF // 16, scan_best,
                (fill, jnp.full((16,), _BIGI, jnp.int32)))
            m = jnp.max(vb)
            mi = jnp.min(jnp.where(vb == m, vi, _BIGI))

            def rem(q, carry):
                ix = cidx[pl.ds(q * 16, 16)]
                v = cval[pl.ds(q * 16, 16)]
                cval[pl.ds(q * 16, 16)] = jnp.where(ix == mi, _FILL, v)
                return carry

            lax.fori_loop(0, _CBUF // 16, rem, jnp.int32(0))
            pos = lb * _BEAM + r
            ovec = jnp.where(lane == pos, m, ovec)
            oivec = jnp.where(lane == pos, mi, oivec)

    outv[...] = ovec
    outi[...] = oivec
    n_out = _BPW * _BEAM
    pltpu.sync_copy(outv.at[pl.ds(0, n_out)], vals_hbm.at[pl.ds(wid * n_out, n_out)])
    pltpu.sync_copy(outi.at[pl.ds(0, n_out)], idx_hbm.at[pl.ds(wid * n_out, n_out)])


def _make_sc_topk():
    return functools.partial(
        pl.kernel,
        mesh=plsc.VectorSubcoreMesh(core_axis_name="c", subcore_axis_name="s"),
        compiler_params=pltpu.CompilerParams(needs_layout_passes=False),
        out_type=[
        jax.ShapeDtypeStruct((_B * _BEAM,), jnp.float32),
        jax.ShapeDtypeStruct((_B * _BEAM,), jnp.int32),
    ],
    scratch_types=[
        pltpu.VMEM((_CHUNK,), jnp.float32),
        pltpu.VMEM((_CHUNK,), jnp.float32),
        pltpu.VMEM((_BPW * _NWIN * 16,), jnp.float32),
        pltpu.VMEM((_WIN,), jnp.float32),
        pltpu.VMEM((_CBUF,), jnp.float32),
        pltpu.VMEM((_CBUF,), jnp.int32),
        pltpu.VMEM((16,), jnp.float32),
        pltpu.VMEM((16,), jnp.float32),
        pltpu.VMEM((16,), jnp.int32),
            pltpu.SemaphoreType.DMA,
            pltpu.SemaphoreType.DMA,
        ],
    )(_sc_topk_body)


_CHL = 1024
_NCHL = 98
_TAIL_A = 97 * _CHL
_TFILL = -3.0e38
_G = 4
_GX = 8
_BIG = 2 ** 30


def _scan_body(x_ref, alive_ref, cmax_ref):
    i0 = pl.program_id(0) * _G
    br = jax.lax.broadcasted_iota(jnp.int32, (_BEAM, 1), 0)
    for g in range(_G):
        s_list = [alive_ref[i0 + g, k] for k in range(_BEAM)]
        s_col = jnp.where(br == 0, s_list[0],
                          jnp.where(br == 1, s_list[1],
                                    jnp.where(br == 2, s_list[2], s_list[3])))
        cms = []
        for c in range(_NCHL):
            a = c * _CHL
            if c < _NCHL - 1:
                w = x_ref[g, :, a:a + 128]
                for t in range(1, _CHL // 128):
                    w = jnp.maximum(w, x_ref[g, :, a + 128 * t:a + 128 * (t + 1)])
                cm = jnp.max(w, axis=1, keepdims=True)
            else:
                w = x_ref[g, :, a:a + 128]
                for t in range(1, (_V - _TAIL_A) // 128):
                    w = jnp.maximum(w, x_ref[g, :, a + 128 * t:a + 128 * (t + 1)])
                cm = jnp.maximum(
                    jnp.max(w, axis=1, keepdims=True),
                    jnp.max(x_ref[g, :, _TAIL_A + ((_V - _TAIL_A) // 128) * 128:_V],
                            axis=1, keepdims=True))
            cms.append(cm)
        cmat = jnp.concatenate(cms, axis=1) + s_col
        cmax_ref[0, g:g + 1, :] = jnp.max(cmat, axis=0, keepdims=True)


def _chunksel_body(cmax_ref, cand_ref):
    cm = cmax_ref[...]
    cio = jax.lax.broadcasted_iota(jnp.int32, (_B, _NCHL), 1)
    ids = []
    for r in range(_BEAM):
        m = jnp.max(cm, axis=1, keepdims=True)
        cid = jnp.min(jnp.where(cm == m, cio, _BIG), axis=1, keepdims=True)
        ids.append(cid)
        cm = jnp.where(cio == cid, _TFILL, cm)
    cand_ref[...] = jnp.concatenate(ids, axis=1)


def _gx_body(cand_ref, *refs):
    xrefs = refs[:4 * _GX]
    alive_ref = refs[4 * _GX]
    vals_ref, idx_ref = refs[4 * _GX + 1], refs[4 * _GX + 2]
    i0 = pl.program_id(0) * _GX
    kio2 = jax.lax.broadcasted_iota(jnp.int32, (_BEAM, _CHL), 0) * _V
    lio2 = jax.lax.broadcasted_iota(jnp.int32, (_BEAM, _CHL), 1)
    br = jax.lax.broadcasted_iota(jnp.int32, (_BEAM, 1), 0)
    for g in range(_GX):
        b = i0 + g
        s_list = [alive_ref[b, k] for k in range(_BEAM)]
        s_col = jnp.where(br == 0, s_list[0],
                          jnp.where(br == 1, s_list[1],
                                    jnp.where(br == 2, s_list[2], s_list[3])))
        mys = []
        fids = []
        for j in range(4):
            xr = xrefs[g * 4 + j]
            cid = cand_ref[b, j]
            vpos = cid * _CHL + lio2
            yj = xr[0] + s_col
            mys.append(jnp.where(vpos < _V, yj, _TFILL))
            fids.append(kio2 + vpos)
        my = jnp.concatenate(mys, axis=1)
        fidx = jnp.concatenate(fids, axis=1)
        for r in range(_BEAM):
            m = jnp.max(jnp.max(my, axis=1, keepdims=True), axis=0,
                        keepdims=True)
            c1 = jnp.min(jnp.where(my == m, fidx, _BIG), axis=1, keepdims=True)
            sel = jnp.min(c1, axis=0, keepdims=True)
            vals_ref[g, :, r:r + 1] = m
            idx_ref[g, :, r:r + 1] = sel
            my = jnp.where(fidx == sel, _TFILL, my)


def _topk_body(score_ref, alive_ref, vals_ref, idx_ref):
    b = pl.program_id(0)
    x = score_ref[0]
    ridx = jax.lax.broadcasted_iota(jnp.int32, (_SUB, _LANES), 0)
    cidx = jax.lax.broadcasted_iota(jnp.int32, (_SUB, _LANES), 1)
    idx = ridx * _LANES + cidx
    s0 = alive_ref[b, 0]
    s1 = alive_ref[b, 1]
    s2 = alive_ref[b, 2]
    s3 = alive_ref[b, 3]
    add = jnp.where(idx < _V, s0, jnp.where(idx < 2 * _V, s1,
                    jnp.where(idx < 3 * _V, s2, s3)))
    y = x + add
    big = jnp.int32(2 ** 30)
    for r in range(_BEAM):
        m = jnp.max(y)
        sel = jnp.min(jnp.where(y == m, idx, big))
        vals_ref[0, 0, r] = m
        idx_ref[0, 0, r] = sel
        y = jnp.where(idx == sel, _NEG_INF, y)


def _finish_body(vals_ref, idx_ref, hyp_ref, ts_ref, as_ref, fm_ref, tok_ref, hyp_out_ref):
    top_scores = vals_ref[:, 0, :]
    index = idx_ref[:, 0, :]
    tokens = index % _V
    origin = index // _V
    hyp = hyp_ref[...]
    lane = jax.lax.broadcasted_iota(jnp.int32, (_B, _BEAM * _CUR), 1)
    grp = lane // _CUR
    zero32 = jnp.zeros((_B, _BEAM * _CUR), jnp.int32)
    origin_e = zero32
    for j in range(_BEAM):
        origin_e = jnp.where(grp == j, origin[:, j:j + 1], origin_e)
    cand = zero32
    for k in range(_BEAM):
        tile_k = jnp.concatenate([hyp[:, k * _CUR:(k + 1) * _CUR]] * _BEAM, axis=1)
        cand = jnp.where(origin_e == k, tile_k, cand)
    flags = (tokens == _EOS).astype(jnp.float32)
    alive_masked = top_scores + flags * _NEG_INF
    finish_masked = top_scores + (1.0 - flags) * _NEG_INF
    iota4 = jax.lax.broadcasted_iota(jnp.int32, (_B, _BEAM), 1)
    am = alive_masked
    new_scores = []
    new_idx = []
    for r in range(_BEAM):
        m = jnp.max(am, axis=1, keepdims=True)
        sel = jnp.min(jnp.where(am == m, iota4, _BEAM), axis=1, keepdims=True)
        new_scores.append(m)
        new_idx.append(sel)
        am = jnp.where(iota4 == sel, _NEG_INF, am)
    alive_scores_new = jnp.concatenate(new_scores, axis=1)
    alive_idx = jnp.concatenate(new_idx, axis=1)
    aidx_e = zero32
    for j in range(_BEAM):
        aidx_e = jnp.where(grp == j, alive_idx[:, j:j + 1], aidx_e)
    new_hyp = zero32
    new_tok = jnp.zeros((_B, _BEAM), jnp.int32)
    for k in range(_BEAM):
        tile_k = jnp.concatenate([cand[:, k * _CUR:(k + 1) * _CUR]] * _BEAM, axis=1)
        new_hyp = jnp.where(aidx_e == k, tile_k, new_hyp)
        new_tok = jnp.where(alive_idx == k, tokens[:, k:k + 1], new_tok)
    ts_ref[...] = top_scores
    as_ref[...] = alive_scores_new
    fm_ref[...] = finish_masked
    tok_ref[...] = tokens
    hyp_out_ref[...] = jnp.concatenate(
        [jnp.concatenate([new_hyp[:, j * _CUR:(j + 1) * _CUR],
                          new_tok[:, j:j + 1]], axis=1)
         for j in range(_BEAM)], axis=1)


def kernel(out, alive_scores, alive_hypotheses):
    cmax = pl.pallas_call(
        _scan_body,
        grid=(_B // _G,),
        in_specs=[
            pl.BlockSpec((_G, _BEAM, _V), lambda b: (b, 0, 0)),
            pl.BlockSpec(memory_space=pltpu.SMEM),
        ],
        out_specs=pl.BlockSpec((1, _G, _NCHL), lambda b: (b, 0, 0)),
        out_shape=jax.ShapeDtypeStruct((_B // _G, _G, _NCHL), jnp.float32),
    )(out, alive_scores)

    cand = pl.pallas_call(
        _chunksel_body,
        out_shape=jax.ShapeDtypeStruct((_B, _BEAM), jnp.int32),
    )(cmax.reshape(_B, _NCHL))

    vals, idx = pl.pallas_call(
        _gx_body,
        grid_spec=pltpu.PrefetchScalarGridSpec(
            num_scalar_prefetch=1,
            grid=(_B // _GX,),
            in_specs=[
                pl.BlockSpec(
                    (1, _BEAM, _CHL),
                    lambda i, cand_pref, g=g, j=j: (
                        i * _GX + g, 0, cand_pref[i * _GX + g, j]))
                for g in range(_GX) for j in range(4)
            ] + [pl.BlockSpec(memory_space=pltpu.SMEM)],
            out_specs=[
                pl.BlockSpec((_GX, 1, _BEAM), lambda i, cand_pref: (i, 0, 0)),
                pl.BlockSpec((_GX, 1, _BEAM), lambda i, cand_pref: (i, 0, 0)),
            ],
        ),
        out_shape=[
            jax.ShapeDtypeStruct((_B, 1, _BEAM), jnp.float32),
            jax.ShapeDtypeStruct((_B, 1, _BEAM), jnp.int32),
        ],
    )(cand, *([out] * (4 * _GX)), alive_scores)

    ts, asn, fm, tok, hyp_new = pl.pallas_call(
        _finish_body,
        out_shape=[
            jax.ShapeDtypeStruct((_B, _BEAM), jnp.float32),
            jax.ShapeDtypeStruct((_B, _BEAM), jnp.float32),
            jax.ShapeDtypeStruct((_B, _BEAM), jnp.float32),
            jax.ShapeDtypeStruct((_B, _BEAM), jnp.int32),
            jax.ShapeDtypeStruct((_B, _BEAM * (_CUR + 1)), jnp.int32),
        ],
    )(vals, idx, alive_hypotheses.reshape(_B, _BEAM * _CUR))
    return (ts, asn, fm, tok, hyp_new.reshape(_B * _BEAM, _CUR + 1))


# speedup vs baseline: 2.9090x; 1.0584x over previous
import functools

import jax
import jax.numpy as jnp
from jax import lax
from jax.experimental import pallas as pl
from jax.experimental.pallas import tpu as pltpu
from jax.experimental.pallas import tpu_sc as plsc

_B = 64
_BEAM = 4
_V = 100000
_CUR = 8
_EOS = 2
_NEG_INF = -1e20
_ROW = _BEAM * _V
_LANES = 128
_SUB = _ROW // _LANES

_NW_WORKERS = 32
_BPW = _B // _NW_WORKERS
_CHUNK = 20000
_NCHUNK = _BPW * _ROW // _CHUNK
_WIN = 2000
_NWIN = _ROW // _WIN
_FILL = -1.0e30
_BIGI = 2 ** 30
_CBUF = 512


def _sc_topk_body(scores_hbm, alive_hbm, vals_hbm, idx_hbm,
                  buf0, buf1, mw, rbuf, cval, cidx, asv, outv, outi,
                  sem0, sem1):
    wid = lax.axis_index("s") * 2 + lax.axis_index("c")
    base = wid * (_BPW * _ROW)

    pltpu.sync_copy(alive_hbm.at[pl.ds(wid * (_BPW * _BEAM), _BPW * _BEAM)],
                    asv.at[pl.ds(0, _BPW * _BEAM)])
    av = asv[...]

    def _chunk_src(c):
        return scores_hbm.at[pl.ds(base + c * _CHUNK, _CHUNK)]

    def _compute_chunk(buf, chunk_i):
        for j in range(_CHUNK // _WIN):
            def inner(t, accs):
                o = j * _WIN + t * 80
                return tuple(
                    jnp.maximum(a, buf[pl.ds(o + 16 * q, 16)])
                    for q, a in enumerate(accs))
            init = tuple(jnp.full((16,), _FILL, jnp.float32) for _ in range(5))
            a0, a1, a2, a3, a4 = lax.fori_loop(0, _WIN // 80, inner, init)
            acc = jnp.maximum(jnp.maximum(jnp.maximum(a0, a1),
                                          jnp.maximum(a2, a3)), a4)
            mw[pl.ds((chunk_i * (_CHUNK // _WIN) + j) * 16, 16)] = acc

    pltpu.async_copy(_chunk_src(0), buf0, sem0)

    def outer(i, carry):
        pltpu.async_copy(_chunk_src(2 * i + 1), buf1, sem1)
        pltpu.make_async_copy(_chunk_src(0), buf0, sem0).wait()
        _compute_chunk(buf0, 2 * i)

        @pl.when(i < _NCHUNK // 2 - 1)
        def _():
            pltpu.async_copy(_chunk_src(2 * i + 2), buf0, sem0)

        pltpu.make_async_copy(_chunk_src(1), buf1, sem1).wait()
        _compute_chunk(buf1, 2 * i + 1)
        return carry

    lax.fori_loop(0, _NCHUNK // 2, outer, jnp.int32(0))

    lane = lax.iota(jnp.int32, 16)
    ovec = jnp.full((16,), _FILL, jnp.float32)
    oivec = jnp.full((16,), 0, jnp.int32)

    for lb in range(_BPW):
        mwbase = lb * _NWIN * 16
        s0 = av[lb * _BEAM + 0]
        s1 = av[lb * _BEAM + 1]
        s2 = av[lb * _BEAM + 2]
        s3 = av[lb * _BEAM + 3]
        wps = _V // _WIN

        def _sk(wi):
            return jnp.where(wi < wps, s0,
                             jnp.where(wi < 2 * wps, s1,
                                       jnp.where(wi < 3 * wps, s2, s3)))

        def tins(wi, carry):
            t1, t2, t3, t4 = carry
            x = mw[pl.ds(mwbase + wi * 16, 16)] + _sk(wi)
            d = jnp.minimum(t1, x)
            t1 = jnp.maximum(t1, x)
            d2 = jnp.minimum(t2, d)
            t2 = jnp.maximum(t2, d)
            d3 = jnp.minimum(t3, d2)
            t3 = jnp.maximum(t3, d2)
            t4 = jnp.maximum(t4, d3)
            return (t1, t2, t3, t4)

        fill = jnp.full((16,), _FILL, jnp.float32)
        t1, t2, t3, t4 = lax.fori_loop(0, _NWIN, tins, (fill, fill, fill, fill))

        T = jnp.float32(0)
        for r in range(4):
            mm = jnp.maximum(jnp.maximum(t1, t2), jnp.maximum(t3, t4))
            T = jnp.max(mm)
            if r < 3:
                t1 = jnp.where(t1 == T, _FILL, t1)
                t2 = jnp.where(t2 == T, _FILL, t2)
                t3 = jnp.where(t3 == T, _FILL, t3)
                t4 = jnp.where(t4 == T, _FILL, t4)

        for q in range(_CBUF // 16):
            cval[pl.ds(q * 16, 16)] = fill
            cidx[pl.ds(q * 16, 16)] = jnp.full((16,), _BIGI, jnp.int32)

        def rw(wi, off):
            a = mw[pl.ds(mwbase + wi * 16, 16)]
            sk = _sk(wi)
            wmax = jnp.max(a) + sk

            def do_rescan(off):
                pltpu.sync_copy(
                    scores_hbm.at[pl.ds(base + lb * _ROW + wi * _WIN, _WIN)],
                    rbuf)

                def rv(t, off):
                    y = rbuf[pl.ds(t * 16, 16)] + sk
                    msk = y >= T
                    iv = lane + (wi * _WIN + t * 16)
                    plsc.store_compressed(cval.at[pl.ds(off, 16)], y, mask=msk)
                    plsc.store_compressed(cidx.at[pl.ds(off, 16)], iv, mask=msk)
                    cnt = plsc.all_reduce_population_count(msk)
                    return jnp.minimum(off + jnp.max(cnt),
                                       jnp.int32(_CBUF - 16))

                return lax.fori_loop(0, _WIN // 16, rv, off)

            return lax.cond(wmax >= T, do_rescan, lambda o: o, off)

        lax.fori_loop(0, _NWIN, rw, jnp.int32(0))

        for r in range(4):
            def scan_best(q, carry):
                vb, vi = carry
                v = cval[pl.ds(q * 16, 16)]
                ix = cidx[pl.ds(q * 16, 16)]
                better = (v > vb) | ((v == vb) & (ix < vi))
                return (jnp.where(better, v, vb), jnp.where(better, ix, vi))

            vb, vi = lax.fori_loop(
                0, _CBUF // 16, scan_best,
                (fill, jnp.full((16,), _BIGI, jnp.int32)))
            m = jnp.max(vb)
            mi = jnp.min(jnp.where(vb == m, vi, _BIGI))

            def rem(q, carry):
                ix = cidx[pl.ds(q * 16, 16)]
                v = cval[pl.ds(q * 16, 16)]
                cval[pl.ds(q * 16, 16)] = jnp.where(ix == mi, _FILL, v)
                return carry

            lax.fori_loop(0, _CBUF // 16, rem, jnp.int32(0))
            pos = lb * _BEAM + r
            ovec = jnp.where(lane == pos, m, ovec)
            oivec = jnp.where(lane == pos, mi, oivec)

    outv[...] = ovec
    outi[...] = oivec
    n_out = _BPW * _BEAM
    pltpu.sync_copy(outv.at[pl.ds(0, n_out)], vals_hbm.at[pl.ds(wid * n_out, n_out)])
    pltpu.sync_copy(outi.at[pl.ds(0, n_out)], idx_hbm.at[pl.ds(wid * n_out, n_out)])


def _make_sc_topk():
    return functools.partial(
        pl.kernel,
        mesh=plsc.VectorSubcoreMesh(core_axis_name="c", subcore_axis_name="s"),
        compiler_params=pltpu.CompilerParams(needs_layout_passes=False),
        out_type=[
        jax.ShapeDtypeStruct((_B * _BEAM,), jnp.float32),
        jax.ShapeDtypeStruct((_B * _BEAM,), jnp.int32),
    ],
    scratch_types=[
        pltpu.VMEM((_CHUNK,), jnp.float32),
        pltpu.VMEM((_CHUNK,), jnp.float32),
        pltpu.VMEM((_BPW * _NWIN * 16,), jnp.float32),
        pltpu.VMEM((_WIN,), jnp.float32),
        pltpu.VMEM((_CBUF,), jnp.float32),
        pltpu.VMEM((_CBUF,), jnp.int32),
        pltpu.VMEM((16,), jnp.float32),
        pltpu.VMEM((16,), jnp.float32),
        pltpu.VMEM((16,), jnp.int32),
            pltpu.SemaphoreType.DMA,
            pltpu.SemaphoreType.DMA,
        ],
    )(_sc_topk_body)


_CHL = 512
_NCHL = 196
_TAIL_A = 195 * _CHL
_TFILL = -3.0e38
_G = 4
_GX = 8
_BIG = 2 ** 30


def _scan_body(x_ref, alive_ref, cmax_ref):
    i0 = pl.program_id(0) * _G
    br = jax.lax.broadcasted_iota(jnp.int32, (_BEAM, 1), 0)
    for g in range(_G):
        s_list = [alive_ref[i0 + g, k] for k in range(_BEAM)]
        s_col = jnp.where(br == 0, s_list[0],
                          jnp.where(br == 1, s_list[1],
                                    jnp.where(br == 2, s_list[2], s_list[3])))
        cms = []
        for c in range(_NCHL):
            a = c * _CHL
            if c < _NCHL - 1:
                w = x_ref[g, :, a:a + 128]
                for t in range(1, _CHL // 128):
                    w = jnp.maximum(w, x_ref[g, :, a + 128 * t:a + 128 * (t + 1)])
                cm = jnp.max(w, axis=1, keepdims=True)
            else:
                w = x_ref[g, :, a:a + 128]
                for t in range(1, (_V - _TAIL_A) // 128):
                    w = jnp.maximum(w, x_ref[g, :, a + 128 * t:a + 128 * (t + 1)])
                cm = jnp.maximum(
                    jnp.max(w, axis=1, keepdims=True),
                    jnp.max(x_ref[g, :, _TAIL_A + ((_V - _TAIL_A) // 128) * 128:_V],
                            axis=1, keepdims=True))
            cms.append(cm)
        cmat = jnp.concatenate(cms, axis=1) + s_col
        cmax_ref[0, g:g + 1, :] = jnp.max(cmat, axis=0, keepdims=True)


def _chunksel_body(cmax_ref, cand_ref):
    cm = cmax_ref[...]
    cio = jax.lax.broadcasted_iota(jnp.int32, (_B, _NCHL), 1)
    ids = []
    for r in range(_BEAM):
        m = jnp.max(cm, axis=1, keepdims=True)
        cid = jnp.min(jnp.where(cm == m, cio, _BIG), axis=1, keepdims=True)
        ids.append(cid)
        cm = jnp.where(cio == cid, _TFILL, cm)
    cand_ref[...] = jnp.concatenate(ids, axis=1)


def _gx_body(cand_ref, *refs):
    xrefs = refs[:4 * _GX]
    alive_ref = refs[4 * _GX]
    vals_ref, idx_ref = refs[4 * _GX + 1], refs[4 * _GX + 2]
    i0 = pl.program_id(0) * _GX
    kio2 = jax.lax.broadcasted_iota(jnp.int32, (_BEAM, _CHL), 0) * _V
    lio2 = jax.lax.broadcasted_iota(jnp.int32, (_BEAM, _CHL), 1)
    br = jax.lax.broadcasted_iota(jnp.int32, (_BEAM, 1), 0)
    for g in range(_GX):
        b = i0 + g
        s_list = [alive_ref[b, k] for k in range(_BEAM)]
        s_col = jnp.where(br == 0, s_list[0],
                          jnp.where(br == 1, s_list[1],
                                    jnp.where(br == 2, s_list[2], s_list[3])))
        mys = []
        fids = []
        for j in range(4):
            xr = xrefs[g * 4 + j]
            cid = cand_ref[b, j]
            vpos = cid * _CHL + lio2
            yj = xr[0] + s_col
            mys.append(jnp.where(vpos < _V, yj, _TFILL))
            fids.append(kio2 + vpos)
        my = jnp.concatenate(mys, axis=1)
        fidx = jnp.concatenate(fids, axis=1)
        for r in range(_BEAM):
            m = jnp.max(jnp.max(my, axis=1, keepdims=True), axis=0,
                        keepdims=True)
            c1 = jnp.min(jnp.where(my == m, fidx, _BIG), axis=1, keepdims=True)
            sel = jnp.min(c1, axis=0, keepdims=True)
            vals_ref[g, :, r:r + 1] = m
            idx_ref[g, :, r:r + 1] = sel
            my = jnp.where(fidx == sel, _TFILL, my)


def _topk_body(score_ref, alive_ref, vals_ref, idx_ref):
    b = pl.program_id(0)
    x = score_ref[0]
    ridx = jax.lax.broadcasted_iota(jnp.int32, (_SUB, _LANES), 0)
    cidx = jax.lax.broadcasted_iota(jnp.int32, (_SUB, _LANES), 1)
    idx = ridx * _LANES + cidx
    s0 = alive_ref[b, 0]
    s1 = alive_ref[b, 1]
    s2 = alive_ref[b, 2]
    s3 = alive_ref[b, 3]
    add = jnp.where(idx < _V, s0, jnp.where(idx < 2 * _V, s1,
                    jnp.where(idx < 3 * _V, s2, s3)))
    y = x + add
    big = jnp.int32(2 ** 30)
    for r in range(_BEAM):
        m = jnp.max(y)
        sel = jnp.min(jnp.where(y == m, idx, big))
        vals_ref[0, 0, r] = m
        idx_ref[0, 0, r] = sel
        y = jnp.where(idx == sel, _NEG_INF, y)


def _finish_body(vals_ref, idx_ref, hyp_ref, ts_ref, as_ref, fm_ref, tok_ref, hyp_out_ref):
    top_scores = vals_ref[:, 0, :]
    index = idx_ref[:, 0, :]
    tokens = index % _V
    origin = index // _V
    hyp = hyp_ref[...]
    lane = jax.lax.broadcasted_iota(jnp.int32, (_B, _BEAM * _CUR), 1)
    grp = lane // _CUR
    zero32 = jnp.zeros((_B, _BEAM * _CUR), jnp.int32)
    origin_e = zero32
    for j in range(_BEAM):
        origin_e = jnp.where(grp == j, origin[:, j:j + 1], origin_e)
    cand = zero32
    for k in range(_BEAM):
        tile_k = jnp.concatenate([hyp[:, k * _CUR:(k + 1) * _CUR]] * _BEAM, axis=1)
        cand = jnp.where(origin_e == k, tile_k, cand)
    flags = (tokens == _EOS).astype(jnp.float32)
    alive_masked = top_scores + flags * _NEG_INF
    finish_masked = top_scores + (1.0 - flags) * _NEG_INF
    iota4 = jax.lax.broadcasted_iota(jnp.int32, (_B, _BEAM), 1)
    am = alive_masked
    new_scores = []
    new_idx = []
    for r in range(_BEAM):
        m = jnp.max(am, axis=1, keepdims=True)
        sel = jnp.min(jnp.where(am == m, iota4, _BEAM), axis=1, keepdims=True)
        new_scores.append(m)
        new_idx.append(sel)
        am = jnp.where(iota4 == sel, _NEG_INF, am)
    alive_scores_new = jnp.concatenate(new_scores, axis=1)
    alive_idx = jnp.concatenate(new_idx, axis=1)
    aidx_e = zero32
    for j in range(_BEAM):
        aidx_e = jnp.where(grp == j, alive_idx[:, j:j + 1], aidx_e)
    new_hyp = zero32
    new_tok = jnp.zeros((_B, _BEAM), jnp.int32)
    for k in range(_BEAM):
        tile_k = jnp.concatenate([cand[:, k * _CUR:(k + 1) * _CUR]] * _BEAM, axis=1)
        new_hyp = jnp.where(aidx_e == k, tile_k, new_hyp)
        new_tok = jnp.where(alive_idx == k, tokens[:, k:k + 1], new_tok)
    ts_ref[...] = top_scores
    as_ref[...] = alive_scores_new
    fm_ref[...] = finish_masked
    tok_ref[...] = tokens
    hyp_out_ref[...] = jnp.concatenate(
        [jnp.concatenate([new_hyp[:, j * _CUR:(j + 1) * _CUR],
                          new_tok[:, j:j + 1]], axis=1)
         for j in range(_BEAM)], axis=1)


def kernel(out, alive_scores, alive_hypotheses):
    cmax = pl.pallas_call(
        _scan_body,
        grid=(_B // _G,),
        in_specs=[
            pl.BlockSpec((_G, _BEAM, _V), lambda b: (b, 0, 0)),
            pl.BlockSpec(memory_space=pltpu.SMEM),
        ],
        out_specs=pl.BlockSpec((1, _G, _NCHL), lambda b: (b, 0, 0)),
        out_shape=jax.ShapeDtypeStruct((_B // _G, _G, _NCHL), jnp.float32),
    )(out, alive_scores)

    cand = pl.pallas_call(
        _chunksel_body,
        out_shape=jax.ShapeDtypeStruct((_B, _BEAM), jnp.int32),
    )(cmax.reshape(_B, _NCHL))

    vals, idx = pl.pallas_call(
        _gx_body,
        grid_spec=pltpu.PrefetchScalarGridSpec(
            num_scalar_prefetch=1,
            grid=(_B // _GX,),
            in_specs=[
                pl.BlockSpec(
                    (1, _BEAM, _CHL),
                    lambda i, cand_pref, g=g, j=j: (
                        i * _GX + g, 0, cand_pref[i * _GX + g, j]))
                for g in range(_GX) for j in range(4)
            ] + [pl.BlockSpec(memory_space=pltpu.SMEM)],
            out_specs=[
                pl.BlockSpec((_GX, 1, _BEAM), lambda i, cand_pref: (i, 0, 0)),
                pl.BlockSpec((_GX, 1, _BEAM), lambda i, cand_pref: (i, 0, 0)),
            ],
        ),
        out_shape=[
            jax.ShapeDtypeStruct((_B, 1, _BEAM), jnp.float32),
            jax.ShapeDtypeStruct((_B, 1, _BEAM), jnp.int32),
        ],
    )(cand, *([out] * (4 * _GX)), alive_scores)

    ts, asn, fm, tok, hyp_new = pl.pallas_call(
        _finish_body,
        out_shape=[
            jax.ShapeDtypeStruct((_B, _BEAM), jnp.float32),
            jax.ShapeDtypeStruct((_B, _BEAM), jnp.float32),
            jax.ShapeDtypeStruct((_B, _BEAM), jnp.float32),
            jax.ShapeDtypeStruct((_B, _BEAM), jnp.int32),
            jax.ShapeDtypeStruct((_B, _BEAM * (_CUR + 1)), jnp.int32),
        ],
    )(vals, idx, alive_hypotheses.reshape(_B, _BEAM * _CUR))
    return (ts, asn, fm, tok, hyp_new.reshape(_B * _BEAM, _CUR + 1))


# speedup vs baseline: 2.9098x; 1.0003x over previous
import jax
import jax.numpy as jnp
from jax.experimental import pallas as pl
from jax.experimental.pallas import tpu as pltpu

_B = 64
_BEAM = 4
_V = 100000
_CUR = 8
_EOS = 2
_NEG_INF = -1e20

_CHL = 512
_NCHL = 196
_TAIL_A = 195 * _CHL
_TFILL = -3.0e38
_G = 4
_GX = 8
_BIG = 2 ** 30


def _scan_body(x_ref, alive_ref, cmax_ref):
    i0 = pl.program_id(0) * _G
    br = jax.lax.broadcasted_iota(jnp.int32, (_BEAM, 1), 0)
    for g in range(_G):
        s_list = [alive_ref[i0 + g, k] for k in range(_BEAM)]
        s_col = jnp.where(br == 0, s_list[0],
                          jnp.where(br == 1, s_list[1],
                                    jnp.where(br == 2, s_list[2], s_list[3])))
        cms = []
        for c in range(_NCHL):
            a = c * _CHL
            if c < _NCHL - 1:
                w = x_ref[g, :, a:a + 128]
                for t in range(1, _CHL // 128):
                    w = jnp.maximum(w, x_ref[g, :, a + 128 * t:a + 128 * (t + 1)])
                cm = jnp.max(w, axis=1, keepdims=True)
            else:
                w = x_ref[g, :, a:a + 128]
                for t in range(1, (_V - _TAIL_A) // 128):
                    w = jnp.maximum(w, x_ref[g, :, a + 128 * t:a + 128 * (t + 1)])
                cm = jnp.maximum(
                    jnp.max(w, axis=1, keepdims=True),
                    jnp.max(x_ref[g, :, _TAIL_A + ((_V - _TAIL_A) // 128) * 128:_V],
                            axis=1, keepdims=True))
            cms.append(cm)
        cmat = jnp.concatenate(cms, axis=1) + s_col
        cmax_ref[0, g:g + 1, :] = jnp.max(cmat, axis=0, keepdims=True)


def _chunksel_body(cmax_ref, cand_ref):
    cm = cmax_ref[...]
    cio = jax.lax.broadcasted_iota(jnp.int32, (_B, _NCHL), 1)
    ids = []
    for r in range(_BEAM):
        m = jnp.max(cm, axis=1, keepdims=True)
        cid = jnp.min(jnp.where(cm == m, cio, _BIG), axis=1, keepdims=True)
        ids.append(cid)
        cm = jnp.where(cio == cid, _TFILL, cm)
    cand_ref[...] = jnp.concatenate(ids, axis=1)


def _gx_body(cand_ref, *refs):
    xrefs = refs[:4 * _GX]
    alive_ref = refs[4 * _GX]
    vals_ref, idx_ref = refs[4 * _GX + 1], refs[4 * _GX + 2]
    i0 = pl.program_id(0) * _GX
    kio2 = jax.lax.broadcasted_iota(jnp.int32, (_BEAM, _CHL), 0) * _V
    lio2 = jax.lax.broadcasted_iota(jnp.int32, (_BEAM, _CHL), 1)
    br = jax.lax.broadcasted_iota(jnp.int32, (_BEAM, 1), 0)
    for g in range(_GX):
        b = i0 + g
        s_list = [alive_ref[b, k] for k in range(_BEAM)]
        s_col = jnp.where(br == 0, s_list[0],
                          jnp.where(br == 1, s_list[1],
                                    jnp.where(br == 2, s_list[2], s_list[3])))
        mys = []
        fids = []
        for j in range(4):
            xr = xrefs[g * 4 + j]
            cid = cand_ref[b, j]
            vpos = cid * _CHL + lio2
            yj = xr[0] + s_col
            mys.append(jnp.where(vpos < _V, yj, _TFILL))
            fids.append(kio2 + vpos)
        my = jnp.concatenate(mys, axis=1)
        fidx = jnp.concatenate(fids, axis=1)
        for r in range(_BEAM):
            m = jnp.max(jnp.max(my, axis=1, keepdims=True), axis=0,
                        keepdims=True)
            c1 = jnp.min(jnp.where(my == m, fidx, _BIG), axis=1, keepdims=True)
            sel = jnp.min(c1, axis=0, keepdims=True)
            vals_ref[g, :, r:r + 1] = m
            idx_ref[g, :, r:r + 1] = sel
            my = jnp.where(fidx == sel, _TFILL, my)


def _finish_body(vals_ref, idx_ref, hyp_ref, ts_ref, as_ref, fm_ref, tok_ref, hyp_out_ref):
    top_scores = vals_ref[:, 0, :]
    index = idx_ref[:, 0, :]
    tokens = index % _V
    origin = index // _V
    hyp = hyp_ref[...]
    lane = jax.lax.broadcasted_iota(jnp.int32, (_B, _BEAM * _CUR), 1)
    grp = lane // _CUR
    zero32 = jnp.zeros((_B, _BEAM * _CUR), jnp.int32)
    origin_e = zero32
    for j in range(_BEAM):
        origin_e = jnp.where(grp == j, origin[:, j:j + 1], origin_e)
    cand = zero32
    for k in range(_BEAM):
        tile_k = jnp.concatenate([hyp[:, k * _CUR:(k + 1) * _CUR]] * _BEAM, axis=1)
        cand = jnp.where(origin_e == k, tile_k, cand)
    flags = (tokens == _EOS).astype(jnp.float32)
    alive_masked = top_scores + flags * _NEG_INF
    finish_masked = top_scores + (1.0 - flags) * _NEG_INF
    iota4 = jax.lax.broadcasted_iota(jnp.int32, (_B, _BEAM), 1)
    am = alive_masked
    new_scores = []
    new_idx = []
    for r in range(_BEAM):
        m = jnp.max(am, axis=1, keepdims=True)
        sel = jnp.min(jnp.where(am == m, iota4, _BEAM), axis=1, keepdims=True)
        new_scores.append(m)
        new_idx.append(sel)
        am = jnp.where(iota4 == sel, _NEG_INF, am)
    alive_scores_new = jnp.concatenate(new_scores, axis=1)
    alive_idx = jnp.concatenate(new_idx, axis=1)
    aidx_e = zero32
    for j in range(_BEAM):
        aidx_e = jnp.where(grp == j, alive_idx[:, j:j + 1], aidx_e)
    new_hyp = zero32
    new_tok = jnp.zeros((_B, _BEAM), jnp.int32)
    for k in range(_BEAM):
        tile_k = jnp.concatenate([cand[:, k * _CUR:(k + 1) * _CUR]] * _BEAM, axis=1)
        new_hyp = jnp.where(aidx_e == k, tile_k, new_hyp)
        new_tok = jnp.where(alive_idx == k, tokens[:, k:k + 1], new_tok)
    ts_ref[...] = top_scores
    as_ref[...] = alive_scores_new
    fm_ref[...] = finish_masked
    tok_ref[...] = tokens
    hyp_out_ref[...] = jnp.concatenate(
        [jnp.concatenate([new_hyp[:, j * _CUR:(j + 1) * _CUR],
                          new_tok[:, j:j + 1]], axis=1)
         for j in range(_BEAM)], axis=1)


def kernel(out, alive_scores, alive_hypotheses):
    cmax = pl.pallas_call(
        _scan_body,
        grid=(_B // _G,),
        in_specs=[
            pl.BlockSpec((_G, _BEAM, _V), lambda b: (b, 0, 0)),
            pl.BlockSpec(memory_space=pltpu.SMEM),
        ],
        out_specs=pl.BlockSpec((1, _G, _NCHL), lambda b: (b, 0, 0)),
        out_shape=jax.ShapeDtypeStruct((_B // _G, _G, _NCHL), jnp.float32),
    )(out, alive_scores)

    cand = pl.pallas_call(
        _chunksel_body,
        out_shape=jax.ShapeDtypeStruct((_B, _BEAM), jnp.int32),
    )(cmax.reshape(_B, _NCHL))

    vals, idx = pl.pallas_call(
        _gx_body,
        grid_spec=pltpu.PrefetchScalarGridSpec(
            num_scalar_prefetch=1,
            grid=(_B // _GX,),
            in_specs=[
                pl.BlockSpec(
                    (1, _BEAM, _CHL),
                    lambda i, cand_pref, g=g, j=j: (
                        i * _GX + g, 0, cand_pref[i * _GX + g, j]))
                for g in range(_GX) for j in range(4)
            ] + [pl.BlockSpec(memory_space=pltpu.SMEM)],
            out_specs=[
                pl.BlockSpec((_GX, 1, _BEAM), lambda i, cand_pref: (i, 0, 0)),
                pl.BlockSpec((_GX, 1, _BEAM), lambda i, cand_pref: (i, 0, 0)),
            ],
        ),
        out_shape=[
            jax.ShapeDtypeStruct((_B, 1, _BEAM), jnp.float32),
            jax.ShapeDtypeStruct((_B, 1, _BEAM), jnp.int32),
        ],
    )(cand, *([out] * (4 * _GX)), alive_scores)

    ts, asn, fm, tok, hyp_new = pl.pallas_call(
        _finish_body,
        out_shape=[
            jax.ShapeDtypeStruct((_B, _BEAM), jnp.float32),
            jax.ShapeDtypeStruct((_B, _BEAM), jnp.float32),
            jax.ShapeDtypeStruct((_B, _BEAM), jnp.float32),
            jax.ShapeDtypeStruct((_B, _BEAM), jnp.int32),
            jax.ShapeDtypeStruct((_B, _BEAM * (_CUR + 1)), jnp.int32),
        ],
    )(vals, idx, alive_hypotheses.reshape(_B, _BEAM * _CUR))
    return (ts, asn, fm, tok, hyp_new.reshape(_B * _BEAM, _CUR + 1))
